# Initial kernel scaffold; baseline (speedup 1.0000x reference)
import jax
import jax.numpy as jnp
from jax.experimental import pallas as pl


def kernel(x, edge_index, batch, W_emb, b_emb, W0, b0, W1, b1, Wf1, bf1, Wf2, bf2, tri_t, g_mu, g_sig, l_m, l_b, rh_c, rh_r, W_out0, Wr1, br1, Wr2, br2, Wc1, bc1, Wc2, bc2, Wc3, bc3):
    raise NotImplementedError("write your pallas kernel here")


# speedup vs baseline: 5.4608x; 5.4608x over previous
import functools

import jax
import jax.numpy as jnp
from jax import lax
from jax.experimental import pallas as pl
from jax.experimental.pallas import tpu as pltpu
from jax.experimental.pallas import tpu_sc as plsc

N = 10000
E = 320000
D = 128
H = 128
F = 8
FH = 24
G = 8
NC = 10

NCORES = 2
NSUB = 16
NW = NCORES * NSUB
EW = E // NW
K = 80
NCH = EW // K
STRIPE = N // NSUB
SCHUNK = 125
DEGW = 16
FW = 16

NB = 1000
EB = 8000

_f32 = jnp.float32
_i32 = jnp.int32

_MESH = plsc.VectorSubcoreMesh(
    core_axis_name="c", subcore_axis_name="s",
    num_cores=NCORES, num_subcores=NSUB)

_PERM = tuple((kp % 8) * 12 + (kp // 24) * 3 + ((kp % 24) // 8)
              for kp in range(96))


@functools.partial(
    pl.kernel,
    out_type=(jax.ShapeDtypeStruct((NCORES, N, DEGW), _f32),
              jax.ShapeDtypeStruct((NW, NCH, K), _i32)),
    mesh=_MESH,
    compiler_params=pltpu.CompilerParams(use_tc_tiling_on_sc=False, needs_layout_passes=False),
    scratch_types=[
        pltpu.VMEM((NCH, K), _i32),
        pltpu.VMEM((N,), _i32),
        pltpu.VMEM((NCH, K), _i32),
        pltpu.VMEM((K, DEGW), _f32),
        pltpu.VMEM((SCHUNK, DEGW), _f32),
        pltpu.VMEM_SHARED((N, DEGW), _f32),
    ],
)
def _sc_deg(dst3_h, batch_h, e0_h, z16_h, degp_h, eg3_h,
            dstv, batchv, egv, onesv, zb, degtab):
    ci = lax.axis_index("c")
    si = lax.axis_index("s")
    wid = ci * NSUB + si
    pltpu.sync_copy(dst3_h.at[wid], dstv)
    pltpu.sync_copy(batch_h, batchv)
    pltpu.sync_copy(e0_h, onesv)
    pltpu.sync_copy(z16_h, zb)
    base = si * STRIPE
    for j in range(STRIPE // SCHUNK):
        pltpu.sync_copy(zb, degtab.at[pl.ds(base + j * SCHUNK, SCHUNK)])
    plsc.subcore_barrier()

    def chunk(c, carry):
        pltpu.sync_copy(onesv, degtab.at[dstv.at[c]], add=True)
        for j in range(K // 16):
            idx = dstv[c, pl.ds(j * 16, 16)]
            egv[c, pl.ds(j * 16, 16)] = plsc.load_gather(batchv, [idx])
        return carry

    lax.fori_loop(0, NCH, chunk, 0)
    pltpu.sync_copy(egv, eg3_h.at[wid])
    plsc.subcore_barrier()
    for j in range(STRIPE // SCHUNK):
        st = base + j * SCHUNK
        pltpu.sync_copy(degtab.at[pl.ds(st, SCHUNK)],
                        degp_h.at[ci, pl.ds(st, SCHUNK)])


@functools.partial(
    pl.kernel,
    out_type=jax.ShapeDtypeStruct((NCORES, N, H), _f32),
    mesh=_MESH,
    compiler_params=pltpu.CompilerParams(use_tc_tiling_on_sc=False, needs_layout_passes=False),
    scratch_types=[
        pltpu.VMEM((NCH, K), _i32),
        pltpu.VMEM((NCH, K), _i32),
        pltpu.VMEM((K, H), _f32),
        pltpu.VMEM((SCHUNK, H), _f32),
        pltpu.VMEM_SHARED((N, H), _f32),
        pltpu.SemaphoreType.DMA,
    ],
)
def _sc_agg(y_h, src3_h, dst3_h, z128_h, part_h,
            srcv, dstv, rows, zb, acc, sem):
    ci = lax.axis_index("c")
    si = lax.axis_index("s")
    wid = ci * NSUB + si
    pltpu.sync_copy(src3_h.at[wid], srcv)
    pltpu.sync_copy(dst3_h.at[wid], dstv)
    pltpu.sync_copy(z128_h, zb)
    base = si * STRIPE
    for j in range(STRIPE // SCHUNK):
        pltpu.sync_copy(zb, acc.at[pl.ds(base + j * SCHUNK, SCHUNK)])
    plsc.subcore_barrier()

    def chunk(c, carry):
        pltpu.async_copy(y_h.at[srcv.at[c]], rows, sem).wait()
        pltpu.sync_copy(rows, acc.at[dstv.at[c]], add=True)
        return carry

    lax.fori_loop(0, NCH, chunk, 0)
    plsc.subcore_barrier()
    for j in range(STRIPE // SCHUNK):
        st = base + j * SCHUNK
        pltpu.sync_copy(acc.at[pl.ds(st, SCHUNK)],
                        part_h.at[ci, pl.ds(st, SCHUNK)])


@functools.partial(
    pl.kernel,
    out_type=(jax.ShapeDtypeStruct((NW, NCH, K, FW), _f32),
              jax.ShapeDtypeStruct((NW, NCH, K, FW), _f32)),
    mesh=_MESH,
    compiler_params=pltpu.CompilerParams(use_tc_tiling_on_sc=False, needs_layout_passes=False),
    scratch_types=[
        pltpu.VMEM((NCH, K), _i32),
        pltpu.VMEM((NCH, K), _i32),
        pltpu.VMEM((K, FW), _f32),
        pltpu.VMEM((K, FW), _f32),
        pltpu.SemaphoreType.DMA,
        pltpu.SemaphoreType.DMA,
    ],
)
def _sc_fsfd(filtp_h, src3_h, dst3_h, fs4_h, fd4_h,
             srcv, dstv, rows_a, rows_b, sem_a, sem_b):
    ci = lax.axis_index("c")
    si = lax.axis_index("s")
    wid = ci * NSUB + si
    pltpu.sync_copy(src3_h.at[wid], srcv)
    pltpu.sync_copy(dst3_h.at[wid], dstv)

    def chunk(c, carry):
        a = pltpu.async_copy(filtp_h.at[srcv.at[c]], rows_a, sem_a)
        b = pltpu.async_copy(filtp_h.at[dstv.at[c]], rows_b, sem_b)
        a.wait()
        b.wait()
        pltpu.sync_copy(rows_a, fs4_h.at[wid, c])
        pltpu.sync_copy(rows_b, fd4_h.at[wid, c])
        return carry

    lax.fori_loop(0, NCH, chunk, 0)


def _coord_cols(x0, x1, tt, gmu, gsig, lm, lb, rc, rr):
    cols = []
    for j in range(3):
        cols.append(jax.nn.relu(x1 - jnp.abs(x0 - tt[j])))
    s2 = 2.0 * gsig[0] * gsig[0]
    for j in range(3):
        d2 = (x0 - gmu[j, 0]) ** 2 + (x1 - gmu[j, 1]) ** 2
        cols.append(jnp.exp(-d2 / s2))
    for j in range(3):
        cols.append(x0 * lm[j, 0] + x1 * lm[j, 1] + lb[j])
    for j in range(3):
        a = jnp.abs(x0 - rc[j])
        cols.append(1.0 / (1.0 + a)
                    - 1.0 / (1.0 + jnp.abs(jnp.abs(rr[j]) - a)))
    return jnp.concatenate(cols, axis=1)


def _dotT(a, b):
    return lax.dot_general(a, b, (((0,), (0,)), ((), ())),
                           preferred_element_type=_f32)


def _tca0_body(x_ref, da_ref, db_ref, we_ref, be_ref, w0_ref, b0_ref,
               h_ref, y_ref, rv_ref):
    deg = da_ref[:, 0:1] + db_ref[:, 0:1] + 1.0
    r = 1.0 / jnp.sqrt(deg)
    h = jnp.dot(x_ref[...], we_ref[...], preferred_element_type=_f32) \
        + be_ref[0:1, :]
    y = (jnp.dot(h, w0_ref[...], preferred_element_type=_f32)
         + b0_ref[0:1, :]) * r
    h_ref[...] = h
    y_ref[...] = y
    rv_ref[...] = jnp.broadcast_to(r, (r.shape[0], 8))


def _tca_body(h_ref, pa_ref, pb_ref, y1_ref, rv_ref, w0_ref, b0_ref, dt_ref,
              ho_ref, y_ref):
    r = rv_ref[:, 0:1]
    h = h_ref[...] + dt_ref[0] * (r * (pa_ref[...] + pb_ref[...] + y1_ref[...]))
    y = (jnp.dot(h, w0_ref[...], preferred_element_type=_f32)
         + b0_ref[0:1, :]) * r
    ho_ref[...] = h
    y_ref[...] = y


def _tca3_body(h_ref, pa_ref, pb_ref, y1_ref, rv_ref, bt_ref,
               wr1_ref, br1_ref, wr2_ref, br2_ref, dt_ref, g_ref):
    r = rv_ref[:, 0:1]
    h = h_ref[...] + dt_ref[0] * (r * (pa_ref[...] + pb_ref[...] + y1_ref[...]))
    t = jax.nn.relu(jnp.dot(h, wr1_ref[...], preferred_element_type=_f32)
                    + br1_ref[0:1, :])
    rr = jnp.dot(t, wr2_ref[...], preferred_element_type=_f32) + br2_ref[0:1, :]
    iot = lax.broadcasted_iota(_i32, (rr.shape[0], G), 1)
    oh = (bt_ref[...] == iot).astype(_f32)

    @pl.when(pl.program_id(0) == 0)
    def _():
        g_ref[...] = jnp.zeros(g_ref.shape, _f32)

    g_ref[...] += _dotT(oh, rr)


def _tcb_body(pa_ref, pb_ref, y0_ref, rv_ref, wf1_ref, bf1_ref, wf2_ref,
              bf2_ref, tt_ref, gmu_ref, gsig_ref, lm_ref, lb_ref, rc_ref,
              rr_ref, wo_ref, w1_ref, b1_ref, y1_ref, fp_ref):
    r = rv_ref[:, 0:1]
    z1 = r * (pa_ref[...] + pb_ref[...] + y0_ref[...])
    t1 = jax.nn.relu(jnp.dot(z1, wf1_ref[...], preferred_element_type=_f32)
                     + bf1_ref[0:1, :])
    filt = jnp.dot(t1, wf2_ref[...], preferred_element_type=_f32) \
        + bf2_ref[0:1, :]
    c96 = _coord_cols(filt, filt, tt_ref, gmu_ref, gsig_ref, lm_ref, lb_ref,
                      rc_ref, rr_ref)
    h_out = z1 + jnp.dot(c96, wo_ref[...], preferred_element_type=_f32)
    y1 = (jnp.dot(h_out, w1_ref[...], preferred_element_type=_f32)
          + b1_ref[0:1, :]) * r
    y1_ref[...] = y1
    fp_ref[...] = jnp.concatenate(
        [filt, jnp.zeros((filt.shape[0], FW - F), _f32)], axis=1)


def _tcc_body(fs_ref, fd_ref, eg_ref, tt_ref, gmu_ref, gsig_ref, lm_ref,
              lb_ref, rc_ref, rr_ref, sc_ref):
    fs = fs_ref[:, 0:F]
    fd = fd_ref[:, 0:F]
    x0 = jnp.minimum(fs, fd)
    x1 = jnp.maximum(fs, fd)
    c96 = _coord_cols(x0, x1, tt_ref, gmu_ref, gsig_ref, lm_ref, lb_ref,
                      rc_ref, rr_ref)
    iot = lax.broadcasted_iota(_i32, (c96.shape[0], G), 1)
    oh = (eg_ref[...] == iot).astype(_f32)
    s = _dotT(oh, c96)
    cnt = _dotT(oh, jnp.ones((c96.shape[0], 1), _f32))
    upd = jnp.concatenate([s, cnt, jnp.zeros((G, 31), _f32)], axis=1)

    @pl.when(pl.program_id(0) == 0)
    def _():
        sc_ref[...] = jnp.zeros(sc_ref.shape, _f32)

    sc_ref[...] += upd


def _tcf_body(g_ref, s0_ref, s1_ref, s2_ref, wc1_ref, bc1_ref, wc2_ref,
              bc2_ref, wc3_ref, bc3_ref, out_ref):
    ph = jnp.zeros((G, 96), _f32)
    for s_ref in (s0_ref, s1_ref, s2_ref):
        ph = ph + s_ref[:, 0:96] / (s_ref[:, 96:97] + 1e-6)
    ph = ph * (1.0 / 3.0)
    z = jnp.concatenate([g_ref[...], ph], axis=1)
    z = jax.nn.relu(jnp.dot(z, wc1_ref[...], preferred_element_type=_f32)
                    + bc1_ref[0:1, :])
    z = jax.nn.relu(jnp.dot(z, wc2_ref[...], preferred_element_type=_f32)
                    + bc2_ref[0:1, :])
    out_ref[...] = jnp.dot(z, wc3_ref[...], preferred_element_type=_f32) \
        + bc3_ref[0:1, :]


def _node_spec(w):
    return pl.BlockSpec((NB, w), lambda i: (i, 0))


def _full_spec(shape):
    return pl.BlockSpec(shape, lambda i: tuple(0 for _ in shape))


_SMEM = pl.BlockSpec(memory_space=pltpu.SMEM)
_GRID_N = N // NB
_GRID_E = E // EB

_COORD_SPECS = [_SMEM] * 7


def _nshape(w):
    return jax.ShapeDtypeStruct((N, w), _f32)


_tca0 = pl.pallas_call(
    _tca0_body,
    grid=(_GRID_N,),
    in_specs=[_node_spec(D), _node_spec(DEGW), _node_spec(DEGW),
              _full_spec((D, H)), _full_spec((8, H)),
              _full_spec((H, H)), _full_spec((8, H))],
    out_specs=[_node_spec(H), _node_spec(H), _node_spec(8)],
    out_shape=[_nshape(H), _nshape(H), _nshape(8)],
)

_tca = pl.pallas_call(
    _tca_body,
    grid=(_GRID_N,),
    in_specs=[_node_spec(H), _node_spec(H), _node_spec(H), _node_spec(H),
              _node_spec(8), _full_spec((H, H)), _full_spec((8, H)), _SMEM],
    out_specs=[_node_spec(H), _node_spec(H)],
    out_shape=[_nshape(H), _nshape(H)],
)

_tca3 = pl.pallas_call(
    _tca3_body,
    grid=(_GRID_N,),
    in_specs=[_node_spec(H), _node_spec(H), _node_spec(H), _node_spec(H),
              _node_spec(8), pl.BlockSpec((NB, 1), lambda i: (i, 0)),
              _full_spec((H, 2 * H)), _full_spec((8, 2 * H)),
              _full_spec((2 * H, H)), _full_spec((8, H)), _SMEM],
    out_specs=pl.BlockSpec((G, H), lambda i: (0, 0)),
    out_shape=jax.ShapeDtypeStruct((G, H), _f32),
)

_tcb = pl.pallas_call(
    _tcb_body,
    grid=(_GRID_N,),
    in_specs=[_node_spec(H), _node_spec(H), _node_spec(H), _node_spec(8),
              _full_spec((H, FH)), _full_spec((8, FH)),
              _full_spec((FH, F)), _full_spec((8, F))]
    + _COORD_SPECS
    + [_full_spec((96, H)), _full_spec((H, H)), _full_spec((8, H))],
    out_specs=[_node_spec(H), _node_spec(FW)],
    out_shape=[_nshape(H), _nshape(FW)],
)

_tcc = pl.pallas_call(
    _tcc_body,
    grid=(_GRID_E,),
    in_specs=[pl.BlockSpec((EB, FW), lambda i: (i, 0)),
              pl.BlockSpec((EB, FW), lambda i: (i, 0)),
              pl.BlockSpec((EB, 1), lambda i: (i, 0))]
    + _COORD_SPECS,
    out_specs=pl.BlockSpec((G, 128), lambda i: (0, 0)),
    out_shape=jax.ShapeDtypeStruct((G, 128), _f32),
)

def _full0(shape):
    return pl.BlockSpec(shape, lambda: tuple(0 for _ in shape))


_tcf = pl.pallas_call(
    _tcf_body,
    in_specs=[_full0((G, H)), _full0((G, 128)), _full0((G, 128)),
              _full0((G, 128)),
              _full0((H + 96, H // 2)), _full0((8, H // 2)),
              _full0((H // 2, H // 4)), _full0((8, H // 4)),
              _full0((H // 4, NC)), _full0((8, NC))],
    out_specs=_full0((G, NC)),
    out_shape=jax.ShapeDtypeStruct((G, NC), _f32),
)


def kernel(x, edge_index, batch, W_emb, b_emb, W0, b0, W1, b1, Wf1, bf1,
           Wf2, bf2, tri_t, g_mu, g_sig, l_m, l_b, rh_c, rh_r, W_out0,
           Wr1, br1, Wr2, br2, Wc1, bc1, Wc2, bc2, Wc3, bc3):
    src3 = edge_index[0].astype(_i32).reshape(NW, NCH, K)
    dst3 = edge_index[1].astype(_i32).reshape(NW, NCH, K)
    batch_i = batch.astype(_i32)

    ts = jnp.linspace(0.0, 1.0, 4)
    w0h = W0[1:]
    perm = jnp.asarray(_PERM, dtype=_i32)
    wo_perm = W_out0[perm]
    wc1p = jnp.concatenate([Wc1[:H], Wc1[H:][perm]], axis=0)
    tile8 = lambda b: jnp.tile(b.reshape(1, -1), (8, 1))
    gsig1 = g_sig.reshape(1)
    cps = (tri_t, g_mu, gsig1, l_m, l_b, rh_c, rh_r)

    e0 = jnp.zeros((K, DEGW), _f32).at[:, 0].set(1.0)
    z16 = jnp.zeros((SCHUNK, DEGW), _f32)
    z128 = jnp.zeros((SCHUNK, H), _f32)

    degp, eg3 = _sc_deg(dst3, batch_i, e0, z16)
    eg2 = eg3.reshape(E, 1)

    h, y0, rv = _tca0(x, degp[0], degp[1], W_emb, tile8(b_emb), w0h,
                      tile8(b0 + ts[0] * W0[0]))

    scs = []
    g = None
    for i in range(3):
        part0 = _sc_agg(y0, src3, dst3, z128)
        y1, filtp = _tcb(part0[0], part0[1], y0, rv, Wf1, tile8(bf1), Wf2,
                         tile8(bf2), *cps, wo_perm, W1, tile8(b1))
        fs4, fd4 = _sc_fsfd(filtp, src3, dst3)
        scs.append(_tcc(fs4.reshape(E, FW), fd4.reshape(E, FW), eg2, *cps))
        part1 = _sc_agg(y1, src3, dst3, z128)
        dtv = (ts[i + 1] - ts[i]).reshape(1)
        if i < 2:
            h, y0 = _tca(h, part1[0], part1[1], y1, rv, w0h,
                         tile8(b0 + ts[i + 1] * W0[0]), dtv)
        else:
            g = _tca3(h, part1[0], part1[1], y1, rv, batch_i.reshape(N, 1),
                      Wr1, tile8(br1), Wr2, tile8(br2), dtv)

    return _tcf(g, scs[0], scs[1], scs[2], wc1p, tile8(bc1), Wc2,
                tile8(bc2), Wc3, tile8(bc3))


# speedup vs baseline: 5.7817x; 1.0588x over previous
import functools

import jax
import jax.numpy as jnp
from jax import lax
from jax.experimental import pallas as pl
from jax.experimental.pallas import tpu as pltpu
from jax.experimental.pallas import tpu_sc as plsc

N = 10000
E = 320000
D = 128
H = 128
HC = H // 2
F = 8
FH = 24
G = 8
NC = 10

NCORES = 2
NSUB = 16
NW = NCORES * NSUB
EW = E // NW
K = 80
NCH = EW // K
KA = 100
NCA = EW // KA
ES = E // NSUB
NCS = ES // KA
NBUF = 4
STRIPE = N // NSUB
SCHUNK = 125
DEGW = 16
FW = 16

NB = 1000
EB = 8000

_f32 = jnp.float32
_i32 = jnp.int32

_MESH = plsc.VectorSubcoreMesh(
    core_axis_name="c", subcore_axis_name="s",
    num_cores=NCORES, num_subcores=NSUB)

_SC_PARAMS = pltpu.CompilerParams(
    use_tc_tiling_on_sc=False, needs_layout_passes=False)

_PERM = tuple((kp % 8) * 12 + (kp // 24) * 3 + ((kp % 24) // 8)
              for kp in range(96))


@functools.partial(
    pl.kernel,
    out_type=(jax.ShapeDtypeStruct((NCORES, N, DEGW), _f32),
              jax.ShapeDtypeStruct((NW, NCH, K), _i32)),
    mesh=_MESH,
    compiler_params=_SC_PARAMS,
    scratch_types=[
        pltpu.VMEM((NCH, K), _i32),
        pltpu.VMEM((N,), _i32),
        pltpu.VMEM((NCH, K), _i32),
        pltpu.VMEM((K, DEGW), _f32),
        pltpu.VMEM((SCHUNK, DEGW), _f32),
        pltpu.VMEM_SHARED((N, DEGW), _f32),
        pltpu.SemaphoreType.DMA,
    ],
)
def _sc_deg(dst3_h, batch_h, e0_h, z16_h, degp_h, eg3_h,
            dstv, batchv, egv, onesv, zb, degtab, dsem):
    ci = lax.axis_index("c")
    si = lax.axis_index("s")
    wid = ci * NSUB + si
    pltpu.sync_copy(dst3_h.at[wid], dstv)
    pltpu.sync_copy(batch_h, batchv)
    pltpu.sync_copy(e0_h, onesv)
    pltpu.sync_copy(z16_h, zb)
    base = si * STRIPE
    for j in range(STRIPE // SCHUNK):
        pltpu.sync_copy(zb, degtab.at[pl.ds(base + j * SCHUNK, SCHUNK)])
    plsc.subcore_barrier()

    def chunk(c, carry):
        pltpu.async_copy(onesv, degtab.at[dstv.at[c]], dsem, add=True)

        @pl.when(c >= NBUF)
        def _():
            pltpu.make_async_copy(onesv, degtab.at[dstv.at[c]], dsem).wait()

        for j in range(K // 16):
            idx = dstv[c, pl.ds(j * 16, 16)]
            egv[c, pl.ds(j * 16, 16)] = plsc.load_gather(batchv, [idx])
        return carry

    lax.fori_loop(0, NCH, chunk, 0)
    for _ in range(NBUF):
        pltpu.make_async_copy(onesv, degtab.at[dstv.at[0]], dsem).wait()
    pltpu.sync_copy(egv, eg3_h.at[wid])
    plsc.subcore_barrier()
    for j in range(STRIPE // SCHUNK):
        st = base + j * SCHUNK
        pltpu.sync_copy(degtab.at[pl.ds(st, SCHUNK)],
                        degp_h.at[ci, pl.ds(st, SCHUNK)])


@functools.partial(
    pl.kernel,
    out_type=jax.ShapeDtypeStruct((NCORES, N, HC), _f32),
    mesh=_MESH,
    compiler_params=_SC_PARAMS,
    scratch_types=[
        pltpu.VMEM((NCS, KA), _i32),
        pltpu.VMEM((NCS, KA), _i32),
        [pltpu.VMEM((KA, HC), _f32)] * NBUF,
        pltpu.VMEM((SCHUNK, HC), _f32),
        pltpu.VMEM_SHARED((N, HC), _f32),
        [pltpu.SemaphoreType.DMA] * NBUF,
        [pltpu.SemaphoreType.DMA] * NBUF,
    ],
)
def _sc_agg(y2_h, src2_h, dst2_h, z64_h, part_h,
            srcv, dstv, rows, zb, acc, gsem, ssem):
    ci = lax.axis_index("c")
    si = lax.axis_index("s")
    yc_h = y2_h.at[ci]
    pltpu.sync_copy(src2_h.at[si], srcv)
    pltpu.sync_copy(dst2_h.at[si], dstv)
    pltpu.sync_copy(z64_h, zb)
    base = si * STRIPE
    for j in range(STRIPE // SCHUNK):
        pltpu.sync_copy(zb, acc.at[pl.ds(base + j * SCHUNK, SCHUNK)])
    plsc.subcore_barrier()

    for b in range(2):
        pltpu.async_copy(yc_h.at[srcv.at[b]], rows[b], gsem[b])

    def group(g, carry):
        for b in range(NBUF):
            c = g * NBUF + b
            b2 = (b + 2) % NBUF
            pltpu.make_async_copy(yc_h.at[srcv.at[c]], rows[b], gsem[b]).wait()

            @pl.when(c >= 2)
            def _():
                pltpu.make_async_copy(rows[b2], acc.at[dstv.at[c]],
                                      ssem[b2]).wait()

            @pl.when(c + 2 < NCS)
            def _():
                pltpu.async_copy(yc_h.at[srcv.at[c + 2]], rows[b2], gsem[b2])

            pltpu.async_copy(rows[b], acc.at[dstv.at[c]], ssem[b], add=True)
        return carry

    lax.fori_loop(0, NCS // NBUF, group, 0)
    for b in ((NCS - 2) % NBUF, (NCS - 1) % NBUF):
        pltpu.make_async_copy(rows[b], acc.at[dstv.at[0]], ssem[b]).wait()
    plsc.subcore_barrier()
    for j in range(STRIPE // SCHUNK):
        st = base + j * SCHUNK
        pltpu.sync_copy(acc.at[pl.ds(st, SCHUNK)],
                        part_h.at[ci, pl.ds(st, SCHUNK)])


@functools.partial(
    pl.kernel,
    out_type=(jax.ShapeDtypeStruct((NW, NCA, KA, FW), _f32),
              jax.ShapeDtypeStruct((NW, NCA, KA, FW), _f32)),
    mesh=_MESH,
    compiler_params=_SC_PARAMS,
    scratch_types=[
        pltpu.VMEM((NCA, KA), _i32),
        pltpu.VMEM((NCA, KA), _i32),
        [pltpu.VMEM((KA, FW), _f32)] * NBUF,
        [pltpu.VMEM((KA, FW), _f32)] * NBUF,
        [pltpu.SemaphoreType.DMA] * NBUF,
        [pltpu.SemaphoreType.DMA] * NBUF,
        [pltpu.SemaphoreType.DMA] * NBUF,
        [pltpu.SemaphoreType.DMA] * NBUF,
    ],
)
def _sc_fsfd(filtp_h, src3_h, dst3_h, fs4_h, fd4_h,
             srcv, dstv, ra, rb, gsa, gsb, wsa, wsb):
    ci = lax.axis_index("c")
    si = lax.axis_index("s")
    wid = ci * NSUB + si
    pltpu.sync_copy(src3_h.at[wid], srcv)
    pltpu.sync_copy(dst3_h.at[wid], dstv)
    for b in range(2):
        pltpu.async_copy(filtp_h.at[srcv.at[b]], ra[b], gsa[b])
        pltpu.async_copy(filtp_h.at[dstv.at[b]], rb[b], gsb[b])

    def group(g, carry):
        for b in range(NBUF):
            c = g * NBUF + b
            b2 = (b + 2) % NBUF
            pltpu.make_async_copy(filtp_h.at[srcv.at[c]], ra[b], gsa[b]).wait()
            pltpu.make_async_copy(filtp_h.at[dstv.at[c]], rb[b], gsb[b]).wait()

            @pl.when(c >= 2)
            def _():
                pltpu.make_async_copy(ra[b2], fs4_h.at[wid, c], wsa[b2]).wait()
                pltpu.make_async_copy(rb[b2], fd4_h.at[wid, c], wsb[b2]).wait()

            @pl.when(c + 2 < NCA)
            def _():
                pltpu.async_copy(filtp_h.at[srcv.at[c + 2]], ra[b2], gsa[b2])
                pltpu.async_copy(filtp_h.at[dstv.at[c + 2]], rb[b2], gsb[b2])

            pltpu.async_copy(ra[b], fs4_h.at[wid, c], wsa[b])
            pltpu.async_copy(rb[b], fd4_h.at[wid, c], wsb[b])
        return carry

    lax.fori_loop(0, NCA // NBUF, group, 0)
    for b in ((NCA - 2) % NBUF, (NCA - 1) % NBUF):
        pltpu.make_async_copy(ra[b], fs4_h.at[wid, 0], wsa[b]).wait()
        pltpu.make_async_copy(rb[b], fd4_h.at[wid, 0], wsb[b]).wait()


def _coord_cols(x0, x1, tt, gmu, gsig, lm, lb, rc, rr):
    cols = []
    for j in range(3):
        cols.append(jax.nn.relu(x1 - jnp.abs(x0 - tt[j])))
    s2 = 2.0 * gsig[0] * gsig[0]
    for j in range(3):
        d2 = (x0 - gmu[j, 0]) ** 2 + (x1 - gmu[j, 1]) ** 2
        cols.append(jnp.exp(-d2 / s2))
    for j in range(3):
        cols.append(x0 * lm[j, 0] + x1 * lm[j, 1] + lb[j])
    for j in range(3):
        a = jnp.abs(x0 - rc[j])
        cols.append(1.0 / (1.0 + a)
                    - 1.0 / (1.0 + jnp.abs(jnp.abs(rr[j]) - a)))
    return jnp.concatenate(cols, axis=1)


def _dotT(a, b):
    return lax.dot_general(a, b, (((0,), (0,)), ((), ())),
                           preferred_element_type=_f32)


def _split_y(y, ya_ref, yb_ref):
    ya_ref[...] = y[:, 0:HC]
    yb_ref[...] = y[:, HC:H]


def _tca0_body(x_ref, da_ref, db_ref, we_ref, be_ref, w0_ref, b0_ref,
               h_ref, ya_ref, yb_ref, rv_ref):
    deg = da_ref[:, 0:1] + db_ref[:, 0:1] + 1.0
    r = 1.0 / jnp.sqrt(deg)
    h = jnp.dot(x_ref[...], we_ref[...], preferred_element_type=_f32) \
        + be_ref[0:1, :]
    y = (jnp.dot(h, w0_ref[...], preferred_element_type=_f32)
         + b0_ref[0:1, :]) * r
    h_ref[...] = h
    _split_y(y, ya_ref, yb_ref)
    rv_ref[...] = jnp.broadcast_to(r, (r.shape[0], 8))


def _gcn_out(pa_ref, pb_ref, ya_ref, yb_ref, r):
    return r * jnp.concatenate(
        [pa_ref[...] + ya_ref[...], pb_ref[...] + yb_ref[...]], axis=1)


def _tca_body(h_ref, pa_ref, pb_ref, ya_ref, yb_ref, rv_ref, w0_ref, b0_ref,
              dt_ref, ho_ref, yoa_ref, yob_ref):
    r = rv_ref[:, 0:1]
    h = h_ref[...] + dt_ref[0] * _gcn_out(pa_ref, pb_ref, ya_ref, yb_ref, r)
    y = (jnp.dot(h, w0_ref[...], preferred_element_type=_f32)
         + b0_ref[0:1, :]) * r
    ho_ref[...] = h
    _split_y(y, yoa_ref, yob_ref)


def _tca3_body(h_ref, pa_ref, pb_ref, ya_ref, yb_ref, rv_ref, bt_ref,
               wr1_ref, br1_ref, wr2_ref, br2_ref, dt_ref, g_ref):
    r = rv_ref[:, 0:1]
    h = h_ref[...] + dt_ref[0] * _gcn_out(pa_ref, pb_ref, ya_ref, yb_ref, r)
    t = jax.nn.relu(jnp.dot(h, wr1_ref[...], preferred_element_type=_f32)
                    + br1_ref[0:1, :])
    rr = jnp.dot(t, wr2_ref[...], preferred_element_type=_f32) + br2_ref[0:1, :]
    iot = lax.broadcasted_iota(_i32, (rr.shape[0], G), 1)
    oh = (bt_ref[...] == iot).astype(_f32)

    @pl.when(pl.program_id(0) == 0)
    def _():
        g_ref[...] = jnp.zeros(g_ref.shape, _f32)

    g_ref[...] += _dotT(oh, rr)


def _tcb_body(pa_ref, pb_ref, ya_ref, yb_ref, rv_ref, wf1_ref, bf1_ref,
              wf2_ref, bf2_ref, tt_ref, gmu_ref, gsig_ref, lm_ref, lb_ref,
              rc_ref, rr_ref, wo_ref, w1_ref, b1_ref,
              y1a_ref, y1b_ref, fp_ref):
    r = rv_ref[:, 0:1]
    z1 = _gcn_out(pa_ref, pb_ref, ya_ref, yb_ref, r)
    t1 = jax.nn.relu(jnp.dot(z1, wf1_ref[...], preferred_element_type=_f32)
                     + bf1_ref[0:1, :])
    filt = jnp.dot(t1, wf2_ref[...], preferred_element_type=_f32) \
        + bf2_ref[0:1, :]
    c96 = _coord_cols(filt, filt, tt_ref, gmu_ref, gsig_ref, lm_ref, lb_ref,
                      rc_ref, rr_ref)
    h_out = z1 + jnp.dot(c96, wo_ref[...], preferred_element_type=_f32)
    y1 = (jnp.dot(h_out, w1_ref[...], preferred_element_type=_f32)
          + b1_ref[0:1, :]) * r
    _split_y(y1, y1a_ref, y1b_ref)
    fp_ref[...] = jnp.concatenate(
        [filt, jnp.zeros((filt.shape[0], FW - F), _f32)], axis=1)


def _tcc_body(fs_ref, fd_ref, eg_ref, tt_ref, gmu_ref, gsig_ref, lm_ref,
              lb_ref, rc_ref, rr_ref, sc_ref):
    fs = fs_ref[:, 0:F]
    fd = fd_ref[:, 0:F]
    x0 = jnp.minimum(fs, fd)
    x1 = jnp.maximum(fs, fd)
    c96 = _coord_cols(x0, x1, tt_ref, gmu_ref, gsig_ref, lm_ref, lb_ref,
                      rc_ref, rr_ref)
    iot = lax.broadcasted_iota(_i32, (c96.shape[0], G), 1)
    oh = (eg_ref[...] == iot).astype(_f32)
    s = _dotT(oh, c96)
    cnt = _dotT(oh, jnp.ones((c96.shape[0], 1), _f32))
    upd = jnp.concatenate([s, cnt, jnp.zeros((G, 31), _f32)], axis=1)

    @pl.when(pl.program_id(0) == 0)
    def _():
        sc_ref[...] = jnp.zeros(sc_ref.shape, _f32)

    sc_ref[...] += upd


def _tcf_body(g_ref, s0_ref, s1_ref, s2_ref, wc1_ref, bc1_ref, wc2_ref,
              bc2_ref, wc3_ref, bc3_ref, out_ref):
    ph = jnp.zeros((G, 96), _f32)
    for s_ref in (s0_ref, s1_ref, s2_ref):
        ph = ph + s_ref[:, 0:96] / (s_ref[:, 96:97] + 1e-6)
    ph = ph * (1.0 / 3.0)
    z = jnp.concatenate([g_ref[...], ph], axis=1)
    z = jax.nn.relu(jnp.dot(z, wc1_ref[...], preferred_element_type=_f32)
                    + bc1_ref[0:1, :])
    z = jax.nn.relu(jnp.dot(z, wc2_ref[...], preferred_element_type=_f32)
                    + bc2_ref[0:1, :])
    out_ref[...] = jnp.dot(z, wc3_ref[...], preferred_element_type=_f32) \
        + bc3_ref[0:1, :]


def _node_spec(w):
    return pl.BlockSpec((NB, w), lambda i: (i, 0))


def _full_spec(shape):
    return pl.BlockSpec(shape, lambda i: tuple(0 for _ in shape))


def _full0(shape):
    return pl.BlockSpec(shape, lambda: tuple(0 for _ in shape))


_SMEM = pl.BlockSpec(memory_space=pltpu.SMEM)
_GRID_N = N // NB
_GRID_E = E // EB

_COORD_SPECS = [_SMEM] * 7


def _nshape(w):
    return jax.ShapeDtypeStruct((N, w), _f32)


_tca0 = pl.pallas_call(
    _tca0_body,
    grid=(_GRID_N,),
    in_specs=[_node_spec(D), _node_spec(DEGW), _node_spec(DEGW),
              _full_spec((D, H)), _full_spec((8, H)),
              _full_spec((H, H)), _full_spec((8, H))],
    out_specs=[_node_spec(H), _node_spec(HC), _node_spec(HC), _node_spec(8)],
    out_shape=[_nshape(H), _nshape(HC), _nshape(HC), _nshape(8)],
)

_tca = pl.pallas_call(
    _tca_body,
    grid=(_GRID_N,),
    in_specs=[_node_spec(H), _node_spec(HC), _node_spec(HC), _node_spec(HC),
              _node_spec(HC), _node_spec(8),
              _full_spec((H, H)), _full_spec((8, H)), _SMEM],
    out_specs=[_node_spec(H), _node_spec(HC), _node_spec(HC)],
    out_shape=[_nshape(H), _nshape(HC), _nshape(HC)],
)

_tca3 = pl.pallas_call(
    _tca3_body,
    grid=(_GRID_N,),
    in_specs=[_node_spec(H), _node_spec(HC), _node_spec(HC), _node_spec(HC),
              _node_spec(HC), _node_spec(8),
              pl.BlockSpec((NB, 1), lambda i: (i, 0)),
              _full_spec((H, 2 * H)), _full_spec((8, 2 * H)),
              _full_spec((2 * H, H)), _full_spec((8, H)), _SMEM],
    out_specs=pl.BlockSpec((G, H), lambda i: (0, 0)),
    out_shape=jax.ShapeDtypeStruct((G, H), _f32),
)

_tcb = pl.pallas_call(
    _tcb_body,
    grid=(_GRID_N,),
    in_specs=[_node_spec(HC), _node_spec(HC), _node_spec(HC), _node_spec(HC),
              _node_spec(8),
              _full_spec((H, FH)), _full_spec((8, FH)),
              _full_spec((FH, F)), _full_spec((8, F))]
    + _COORD_SPECS
    + [_full_spec((96, H)), _full_spec((H, H)), _full_spec((8, H))],
    out_specs=[_node_spec(HC), _node_spec(HC), _node_spec(FW)],
    out_shape=[_nshape(HC), _nshape(HC), _nshape(FW)],
)

_tcc = pl.pallas_call(
    _tcc_body,
    grid=(_GRID_E,),
    in_specs=[pl.BlockSpec((EB, FW), lambda i: (i, 0)),
              pl.BlockSpec((EB, FW), lambda i: (i, 0)),
              pl.BlockSpec((EB, 1), lambda i: (i, 0))]
    + _COORD_SPECS,
    out_specs=pl.BlockSpec((G, 128), lambda i: (0, 0)),
    out_shape=jax.ShapeDtypeStruct((G, 128), _f32),
)

_tcf = pl.pallas_call(
    _tcf_body,
    in_specs=[_full0((G, H)), _full0((G, 128)), _full0((G, 128)),
              _full0((G, 128)),
              _full0((H + 96, H // 2)), _full0((8, H // 2)),
              _full0((H // 2, H // 4)), _full0((8, H // 4)),
              _full0((H // 4, NC)), _full0((8, NC))],
    out_specs=_full0((G, NC)),
    out_shape=jax.ShapeDtypeStruct((G, NC), _f32),
)


def kernel(x, edge_index, batch, W_emb, b_emb, W0, b0, W1, b1, Wf1, bf1,
           Wf2, bf2, tri_t, g_mu, g_sig, l_m, l_b, rh_c, rh_r, W_out0,
           Wr1, br1, Wr2, br2, Wc1, bc1, Wc2, bc2, Wc3, bc3):
    src = edge_index[0].astype(_i32)
    dst = edge_index[1].astype(_i32)
    src3 = src.reshape(NW, NCA, KA)
    dst3 = dst.reshape(NW, NCA, KA)
    src2 = src.reshape(NSUB, NCS, KA)
    dst2 = dst.reshape(NSUB, NCS, KA)
    dst3d = dst.reshape(NW, NCH, K)
    batch_i = batch.astype(_i32)

    ts = jnp.linspace(0.0, 1.0, 4)
    w0h = W0[1:]
    perm = jnp.asarray(_PERM, dtype=_i32)
    wo_perm = W_out0[perm]
    wc1p = jnp.concatenate([Wc1[:H], Wc1[H:][perm]], axis=0)
    tile8 = lambda b: jnp.tile(b.reshape(1, -1), (8, 1))
    gsig1 = g_sig.reshape(1)
    cps = (tri_t, g_mu, gsig1, l_m, l_b, rh_c, rh_r)

    e0 = jnp.zeros((K, DEGW), _f32).at[:, 0].set(1.0)
    z16 = jnp.zeros((SCHUNK, DEGW), _f32)
    z64 = jnp.zeros((SCHUNK, HC), _f32)

    degp, eg3 = _sc_deg(dst3d, batch_i, e0, z16)
    eg2 = eg3.reshape(E, 1)

    h, ya, yb, rv = _tca0(x, degp[0], degp[1], W_emb, tile8(b_emb), w0h,
                          tile8(b0 + ts[0] * W0[0]))

    scs = []
    g = None
    for i in range(3):
        part0 = _sc_agg(jnp.stack([ya, yb]), src2, dst2, z64)
        y1a, y1b, filtp = _tcb(part0[0], part0[1], ya, yb, rv, Wf1,
                               tile8(bf1), Wf2, tile8(bf2), *cps, wo_perm,
                               W1, tile8(b1))
        fs4, fd4 = _sc_fsfd(filtp, src3, dst3)
        scs.append(_tcc(fs4.reshape(E, FW), fd4.reshape(E, FW), eg2, *cps))
        part1 = _sc_agg(jnp.stack([y1a, y1b]), src2, dst2, z64)
        dtv = (ts[i + 1] - ts[i]).reshape(1)
        if i < 2:
            h, ya, yb = _tca(h, part1[0], part1[1], y1a, y1b, rv, w0h,
                             tile8(b0 + ts[i + 1] * W0[0]), dtv)
        else:
            g = _tca3(h, part1[0], part1[1], y1a, y1b, rv,
                      batch_i.reshape(N, 1), Wr1, tile8(br1), Wr2,
                      tile8(br2), dtv)

    return _tcf(g, scs[0], scs[1], scs[2], wc1p, tile8(bc1), Wc2,
                tile8(bc2), Wc3, tile8(bc3))


# speedup vs baseline: 8.7782x; 1.5183x over previous
import functools

import jax
import jax.numpy as jnp
from jax import lax
from jax.experimental import pallas as pl
from jax.experimental.pallas import tpu as pltpu
from jax.experimental.pallas import tpu_sc as plsc

N = 10000
E = 320000
D = 128
H = 128
HC = H // 2
F = 8
FH = 24
G = 8
NC = 10

NCORES = 2
NSUB = 16
NW = NCORES * NSUB
EW = E // NW
K = 80
NCH = EW // K
KA = 100
NCA = EW // KA
ES = E // NSUB
NCS = ES // KA
NBUF = 4
STRIPE = N // NSUB
SCHUNK = 125
DEGW = 16
FW = 16

NB = 1000
EB = 8000

_f32 = jnp.float32
_i32 = jnp.int32

_MESH = plsc.VectorSubcoreMesh(
    core_axis_name="c", subcore_axis_name="s",
    num_cores=NCORES, num_subcores=NSUB)

_SC_PARAMS = pltpu.CompilerParams(
    use_tc_tiling_on_sc=False, needs_layout_passes=False)

_PERM = tuple((kp % 8) * 12 + (kp // 24) * 3 + ((kp % 24) // 8)
              for kp in range(96))


@functools.partial(
    pl.kernel,
    out_type=(jax.ShapeDtypeStruct((NCORES, N, DEGW), _f32),
              jax.ShapeDtypeStruct((NW, NCH, K), _i32)),
    mesh=_MESH,
    compiler_params=_SC_PARAMS,
    scratch_types=[
        pltpu.VMEM((NCH, K), _i32),
        pltpu.VMEM((N,), _i32),
        pltpu.VMEM((NCH, K), _i32),
        pltpu.VMEM((K, DEGW), _f32),
        pltpu.VMEM((SCHUNK, DEGW), _f32),
        pltpu.VMEM_SHARED((N, DEGW), _f32),
        pltpu.SemaphoreType.DMA,
    ],
)
def _sc_deg(dst3_h, batch_h, e0_h, z16_h, degp_h, eg3_h,
            dstv, batchv, egv, onesv, zb, degtab, dsem):
    ci = lax.axis_index("c")
    si = lax.axis_index("s")
    wid = ci * NSUB + si
    pltpu.sync_copy(dst3_h.at[wid], dstv)
    pltpu.sync_copy(batch_h, batchv)
    pltpu.sync_copy(e0_h, onesv)
    pltpu.sync_copy(z16_h, zb)
    base = si * STRIPE
    for j in range(STRIPE // SCHUNK):
        pltpu.sync_copy(zb, degtab.at[pl.ds(base + j * SCHUNK, SCHUNK)])
    plsc.subcore_barrier()

    def chunk(c, carry):
        pltpu.async_copy(onesv, degtab.at[dstv.at[c]], dsem, add=True)

        @pl.when(c >= NBUF)
        def _():
            pltpu.make_async_copy(onesv, degtab.at[dstv.at[c]], dsem).wait()

        for j in range(K // 16):
            idx = dstv[c, pl.ds(j * 16, 16)]
            egv[c, pl.ds(j * 16, 16)] = plsc.load_gather(batchv, [idx])
        return carry

    lax.fori_loop(0, NCH, chunk, 0)
    for _ in range(NBUF):
        pltpu.make_async_copy(onesv, degtab.at[dstv.at[0]], dsem).wait()
    pltpu.sync_copy(egv, eg3_h.at[wid])
    plsc.subcore_barrier()
    for j in range(STRIPE // SCHUNK):
        st = base + j * SCHUNK
        pltpu.sync_copy(degtab.at[pl.ds(st, SCHUNK)],
                        degp_h.at[ci, pl.ds(st, SCHUNK)])


@functools.partial(
    pl.kernel,
    out_type=jax.ShapeDtypeStruct((NCORES, N, HC), _f32),
    mesh=_MESH,
    compiler_params=_SC_PARAMS,
    scratch_types=[
        pltpu.VMEM((NCS, KA), _i32),
        pltpu.VMEM((NCS, KA), _i32),
        [pltpu.VMEM((KA, HC), _f32)] * NBUF,
        pltpu.VMEM((SCHUNK, HC), _f32),
        pltpu.VMEM_SHARED((N, HC), _f32),
        [pltpu.SemaphoreType.DMA] * NBUF,
        [pltpu.SemaphoreType.DMA] * NBUF,
    ],
)
def _sc_agg(y2_h, src2_h, dst2_h, z64_h, part_h,
            srcv, dstv, rows, zb, acc, gsem, ssem):
    ci = lax.axis_index("c")
    si = lax.axis_index("s")
    yc_h = y2_h.at[ci]
    pltpu.sync_copy(src2_h.at[si], srcv)
    pltpu.sync_copy(dst2_h.at[si], dstv)
    pltpu.sync_copy(z64_h, zb)
    base = si * STRIPE
    for j in range(STRIPE // SCHUNK):
        pltpu.sync_copy(zb, acc.at[pl.ds(base + j * SCHUNK, SCHUNK)])
    plsc.subcore_barrier()

    for b in range(2):
        pltpu.async_copy(yc_h.at[srcv.at[b]], rows[b], gsem[b])

    def group(g, carry):
        for b in range(NBUF):
            c = g * NBUF + b
            b2 = (b + 2) % NBUF
            pltpu.make_async_copy(yc_h.at[srcv.at[c]], rows[b], gsem[b]).wait()

            @pl.when(c >= 2)
            def _():
                pltpu.make_async_copy(rows[b2], acc.at[dstv.at[c]],
                                      ssem[b2]).wait()

            @pl.when(c + 2 < NCS)
            def _():
                pltpu.async_copy(yc_h.at[srcv.at[c + 2]], rows[b2], gsem[b2])

            pltpu.async_copy(rows[b], acc.at[dstv.at[c]], ssem[b], add=True)
        return carry

    lax.fori_loop(0, NCS // NBUF, group, 0)
    for b in ((NCS - 2) % NBUF, (NCS - 1) % NBUF):
        pltpu.make_async_copy(rows[b], acc.at[dstv.at[0]], ssem[b]).wait()
    plsc.subcore_barrier()
    for j in range(STRIPE // SCHUNK):
        st = base + j * SCHUNK
        pltpu.sync_copy(acc.at[pl.ds(st, SCHUNK)],
                        part_h.at[ci, pl.ds(st, SCHUNK)])


@functools.partial(
    pl.kernel,
    out_type=(jax.ShapeDtypeStruct((E, FW), _f32),
              jax.ShapeDtypeStruct((E, FW), _f32)),
    mesh=_MESH,
    compiler_params=_SC_PARAMS,
    scratch_types=[
        pltpu.VMEM((NCA, KA), _i32),
        pltpu.VMEM((NCA, KA), _i32),
        [pltpu.VMEM((KA, FW), _f32)] * NBUF,
        [pltpu.VMEM((KA, FW), _f32)] * NBUF,
        [pltpu.SemaphoreType.DMA] * NBUF,
        [pltpu.SemaphoreType.DMA] * NBUF,
        [pltpu.SemaphoreType.DMA] * NBUF,
        [pltpu.SemaphoreType.DMA] * NBUF,
    ],
)
def _sc_fsfd(filtp_h, src3_h, dst3_h, fs4_h, fd4_h,
             srcv, dstv, ra, rb, gsa, gsb, wsa, wsb):
    ci = lax.axis_index("c")
    si = lax.axis_index("s")
    wid = ci * NSUB + si
    ebase = wid * EW
    pltpu.sync_copy(src3_h.at[wid], srcv)
    pltpu.sync_copy(dst3_h.at[wid], dstv)
    for b in range(2):
        pltpu.async_copy(filtp_h.at[srcv.at[b]], ra[b], gsa[b])
        pltpu.async_copy(filtp_h.at[dstv.at[b]], rb[b], gsb[b])

    def group(g, carry):
        for b in range(NBUF):
            c = g * NBUF + b
            b2 = (b + 2) % NBUF
            pltpu.make_async_copy(filtp_h.at[srcv.at[c]], ra[b], gsa[b]).wait()
            pltpu.make_async_copy(filtp_h.at[dstv.at[c]], rb[b], gsb[b]).wait()

            dst_sl = pl.ds(ebase + c * KA, KA)

            @pl.when(c >= 2)
            def _():
                pltpu.make_async_copy(ra[b2], fs4_h.at[dst_sl], wsa[b2]).wait()
                pltpu.make_async_copy(rb[b2], fd4_h.at[dst_sl], wsb[b2]).wait()

            @pl.when(c + 2 < NCA)
            def _():
                pltpu.async_copy(filtp_h.at[srcv.at[c + 2]], ra[b2], gsa[b2])
                pltpu.async_copy(filtp_h.at[dstv.at[c + 2]], rb[b2], gsb[b2])

            pltpu.async_copy(ra[b], fs4_h.at[dst_sl], wsa[b])
            pltpu.async_copy(rb[b], fd4_h.at[dst_sl], wsb[b])
        return carry

    lax.fori_loop(0, NCA // NBUF, group, 0)
    for b in ((NCA - 2) % NBUF, (NCA - 1) % NBUF):
        pltpu.make_async_copy(ra[b], fs4_h.at[pl.ds(ebase, KA)], wsa[b]).wait()
        pltpu.make_async_copy(rb[b], fd4_h.at[pl.ds(ebase, KA)], wsb[b]).wait()


def _coord96(x096, x196, p_ref):
    A = p_ref[0:1, :]
    B0 = p_ref[1:2, :]
    B1 = p_ref[2:3, :]
    C0 = p_ref[3:4, :]
    C1 = p_ref[4:5, :]
    C2 = p_ref[5:6, :]
    Dc = p_ref[6:7, :]
    Dr = p_ref[7:8, :]
    mt = p_ref[8:9, :]
    mg = p_ref[9:10, :]
    ml = p_ref[10:11, :]
    mr = p_ref[11:12, :]
    s2r = p_ref[12:13, :]
    tri = jax.nn.relu(x196 - jnp.abs(x096 - A))
    gau = jnp.exp(((x096 - B0) ** 2 + (x196 - B1) ** 2) * s2r)
    lin = x096 * C0 + x196 * C1 + C2
    a = jnp.abs(x096 - Dc)
    rh = 1.0 / (1.0 + a) - 1.0 / (1.0 + jnp.abs(Dr - a))
    return mt * tri + mg * gau + ml * lin + mr * rh


def _dotT(a, b):
    return lax.dot_general(a, b, (((0,), (0,)), ((), ())),
                           preferred_element_type=_f32)


def _split_y(y, ya_ref, yb_ref):
    ya_ref[...] = y[:, 0:HC]
    yb_ref[...] = y[:, HC:H]


def _tca0_body(x_ref, da_ref, db_ref, we_ref, be_ref, w0_ref, b0_ref,
               h_ref, ya_ref, yb_ref, rv_ref):
    deg = da_ref[:, 0:1] + db_ref[:, 0:1] + 1.0
    r = 1.0 / jnp.sqrt(deg)
    h = jnp.dot(x_ref[...], we_ref[...], preferred_element_type=_f32) \
        + be_ref[0:1, :]
    y = (jnp.dot(h, w0_ref[...], preferred_element_type=_f32)
         + b0_ref[0:1, :]) * r
    h_ref[...] = h
    _split_y(y, ya_ref, yb_ref)
    rv_ref[...] = jnp.broadcast_to(r, (r.shape[0], 8))


def _gcn_out(pa_ref, pb_ref, ya_ref, yb_ref, r):
    return r * jnp.concatenate(
        [pa_ref[...] + ya_ref[...], pb_ref[...] + yb_ref[...]], axis=1)


def _tca_body(h_ref, pa_ref, pb_ref, ya_ref, yb_ref, rv_ref, w0_ref, b0_ref,
              dt_ref, ho_ref, yoa_ref, yob_ref):
    r = rv_ref[:, 0:1]
    h = h_ref[...] + dt_ref[0] * _gcn_out(pa_ref, pb_ref, ya_ref, yb_ref, r)
    y = (jnp.dot(h, w0_ref[...], preferred_element_type=_f32)
         + b0_ref[0:1, :]) * r
    ho_ref[...] = h
    _split_y(y, yoa_ref, yob_ref)


def _tca3_body(h_ref, pa_ref, pb_ref, ya_ref, yb_ref, rv_ref, bt_ref,
               wr1_ref, br1_ref, wr2_ref, br2_ref, dt_ref, g_ref):
    r = rv_ref[:, 0:1]
    h = h_ref[...] + dt_ref[0] * _gcn_out(pa_ref, pb_ref, ya_ref, yb_ref, r)
    t = jax.nn.relu(jnp.dot(h, wr1_ref[...], preferred_element_type=_f32)
                    + br1_ref[0:1, :])
    rr = jnp.dot(t, wr2_ref[...], preferred_element_type=_f32) + br2_ref[0:1, :]
    iot = lax.broadcasted_iota(_i32, (rr.shape[0], G), 1)
    oh = (bt_ref[...] == iot).astype(_f32)

    @pl.when(pl.program_id(0) == 0)
    def _():
        g_ref[...] = jnp.zeros(g_ref.shape, _f32)

    g_ref[...] += _dotT(oh, rr)


def _tcb_body(pa_ref, pb_ref, ya_ref, yb_ref, rv_ref, wf1_ref, bf1_ref,
              wf2_ref, bf2_ref, t_ref, p_ref, wo_ref, w1_ref, b1_ref,
              y1a_ref, y1b_ref, fp_ref):
    r = rv_ref[:, 0:1]
    z1 = _gcn_out(pa_ref, pb_ref, ya_ref, yb_ref, r)
    t1 = jax.nn.relu(jnp.dot(z1, wf1_ref[...], preferred_element_type=_f32)
                     + bf1_ref[0:1, :])
    filt = jnp.dot(t1, wf2_ref[...], preferred_element_type=_f32) \
        + bf2_ref[0:1, :]
    f96 = jnp.dot(filt, t_ref[...], preferred_element_type=_f32)
    c96 = _coord96(f96, f96, p_ref)
    h_out = z1 + jnp.dot(c96, wo_ref[...], preferred_element_type=_f32)
    y1 = (jnp.dot(h_out, w1_ref[...], preferred_element_type=_f32)
          + b1_ref[0:1, :]) * r
    _split_y(y1, y1a_ref, y1b_ref)
    fp_ref[...] = jnp.concatenate(
        [filt, jnp.zeros((filt.shape[0], FW - F), _f32)], axis=1)


def _tcc_body(fs_ref, fd_ref, eg_ref, t_ref, p_ref, sc_ref):
    fs = fs_ref[:, 0:F]
    fd = fd_ref[:, 0:F]
    f96s = jnp.dot(fs, t_ref[...], preferred_element_type=_f32)
    f96d = jnp.dot(fd, t_ref[...], preferred_element_type=_f32)
    x096 = jnp.minimum(f96s, f96d)
    x196 = jnp.maximum(f96s, f96d)
    c96 = _coord96(x096, x196, p_ref)
    iot = lax.broadcasted_iota(_i32, (c96.shape[0], G), 1)
    oh = (eg_ref[...] == iot).astype(_f32)
    s = _dotT(oh, c96)
    cnt = _dotT(oh, jnp.ones((c96.shape[0], 1), _f32))
    upd = jnp.concatenate([s, cnt, jnp.zeros((G, 31), _f32)], axis=1)

    @pl.when(pl.program_id(0) == 0)
    def _():
        sc_ref[...] = jnp.zeros(sc_ref.shape, _f32)

    sc_ref[...] += upd


def _tcf_body(g_ref, s0_ref, s1_ref, s2_ref, wc1_ref, bc1_ref, wc2_ref,
              bc2_ref, wc3_ref, bc3_ref, out_ref):
    ph = jnp.zeros((G, 96), _f32)
    for s_ref in (s0_ref, s1_ref, s2_ref):
        ph = ph + s_ref[:, 0:96] / (s_ref[:, 96:97] + 1e-6)
    ph = ph * (1.0 / 3.0)
    z = jnp.concatenate([g_ref[...], ph], axis=1)
    z = jax.nn.relu(jnp.dot(z, wc1_ref[...], preferred_element_type=_f32)
                    + bc1_ref[0:1, :])
    z = jax.nn.relu(jnp.dot(z, wc2_ref[...], preferred_element_type=_f32)
                    + bc2_ref[0:1, :])
    out_ref[...] = jnp.dot(z, wc3_ref[...], preferred_element_type=_f32) \
        + bc3_ref[0:1, :]


def _node_spec(w):
    return pl.BlockSpec((NB, w), lambda i: (i, 0))


def _full_spec(shape):
    return pl.BlockSpec(shape, lambda i: tuple(0 for _ in shape))


def _full0(shape):
    return pl.BlockSpec(shape, lambda: tuple(0 for _ in shape))


_SMEM = pl.BlockSpec(memory_space=pltpu.SMEM)
_GRID_N = N // NB
_GRID_E = E // EB

_COORD_SPECS = [_full_spec((F, 96)), _full_spec((16, 96))]


def _nshape(w):
    return jax.ShapeDtypeStruct((N, w), _f32)


_tca0 = pl.pallas_call(
    _tca0_body,
    grid=(_GRID_N,),
    in_specs=[_node_spec(D), _node_spec(DEGW), _node_spec(DEGW),
              _full_spec((D, H)), _full_spec((8, H)),
              _full_spec((H, H)), _full_spec((8, H))],
    out_specs=[_node_spec(H), _node_spec(HC), _node_spec(HC), _node_spec(8)],
    out_shape=[_nshape(H), _nshape(HC), _nshape(HC), _nshape(8)],
)

_tca = pl.pallas_call(
    _tca_body,
    grid=(_GRID_N,),
    in_specs=[_node_spec(H), _node_spec(HC), _node_spec(HC), _node_spec(HC),
              _node_spec(HC), _node_spec(8),
              _full_spec((H, H)), _full_spec((8, H)), _SMEM],
    out_specs=[_node_spec(H), _node_spec(HC), _node_spec(HC)],
    out_shape=[_nshape(H), _nshape(HC), _nshape(HC)],
)

_tca3 = pl.pallas_call(
    _tca3_body,
    grid=(_GRID_N,),
    in_specs=[_node_spec(H), _node_spec(HC), _node_spec(HC), _node_spec(HC),
              _node_spec(HC), _node_spec(8),
              pl.BlockSpec((NB, 1), lambda i: (i, 0)),
              _full_spec((H, 2 * H)), _full_spec((8, 2 * H)),
              _full_spec((2 * H, H)), _full_spec((8, H)), _SMEM],
    out_specs=pl.BlockSpec((G, H), lambda i: (0, 0)),
    out_shape=jax.ShapeDtypeStruct((G, H), _f32),
)

_tcb = pl.pallas_call(
    _tcb_body,
    grid=(_GRID_N,),
    in_specs=[_node_spec(HC), _node_spec(HC), _node_spec(HC), _node_spec(HC),
              _node_spec(8),
              _full_spec((H, FH)), _full_spec((8, FH)),
              _full_spec((FH, F)), _full_spec((8, F))]
    + _COORD_SPECS
    + [_full_spec((96, H)), _full_spec((H, H)), _full_spec((8, H))],
    out_specs=[_node_spec(HC), _node_spec(HC), _node_spec(FW)],
    out_shape=[_nshape(HC), _nshape(HC), _nshape(FW)],
)

_tcc = pl.pallas_call(
    _tcc_body,
    grid=(_GRID_E,),
    in_specs=[pl.BlockSpec((EB, FW), lambda i: (i, 0)),
              pl.BlockSpec((EB, FW), lambda i: (i, 0)),
              pl.BlockSpec((EB, 1), lambda i: (i, 0))]
    + _COORD_SPECS,
    out_specs=pl.BlockSpec((G, 128), lambda i: (0, 0)),
    out_shape=jax.ShapeDtypeStruct((G, 128), _f32),
)

_tcf = pl.pallas_call(
    _tcf_body,
    in_specs=[_full0((G, H)), _full0((G, 128)), _full0((G, 128)),
              _full0((G, 128)),
              _full0((H + 96, H // 2)), _full0((8, H // 2)),
              _full0((H // 2, H // 4)), _full0((8, H // 4)),
              _full0((H // 4, NC)), _full0((8, NC))],
    out_specs=_full0((G, NC)),
    out_shape=jax.ShapeDtypeStruct((G, NC), _f32),
)


def kernel(x, edge_index, batch, W_emb, b_emb, W0, b0, W1, b1, Wf1, bf1,
           Wf2, bf2, tri_t, g_mu, g_sig, l_m, l_b, rh_c, rh_r, W_out0,
           Wr1, br1, Wr2, br2, Wc1, bc1, Wc2, bc2, Wc3, bc3):
    src = edge_index[0].astype(_i32)
    dst = edge_index[1].astype(_i32)
    src3 = src.reshape(NW, NCA, KA)
    dst3 = dst.reshape(NW, NCA, KA)
    src2 = src.reshape(NSUB, NCS, KA)
    dst2 = dst.reshape(NSUB, NCS, KA)
    dst3d = dst.reshape(NW, NCH, K)
    batch_i = batch.astype(_i32)

    ts = jnp.linspace(0.0, 1.0, 4)
    w0h = W0[1:]
    perm = jnp.asarray(_PERM, dtype=_i32)
    wo_perm = W_out0[perm]
    wc1p = jnp.concatenate([Wc1[:H], Wc1[H:][perm]], axis=0)
    tile8 = lambda b: jnp.tile(b.reshape(1, -1), (8, 1))

    kp = jnp.arange(96)
    tmat = (kp[None, :] % F == jnp.arange(F)[:, None]).astype(_f32)
    jv = (kp % 24) // 8
    fam = kp // 24
    mk = lambda v: v[jv]
    fm = lambda i: (fam == i).astype(_f32)
    s2r = jnp.full((96,), -1.0 / (2.0 * g_sig * g_sig), _f32)
    pmat = jnp.concatenate([
        jnp.stack([mk(tri_t), mk(g_mu[:, 0]), mk(g_mu[:, 1]), mk(l_m[:, 0]),
                   mk(l_m[:, 1]), mk(l_b), mk(rh_c), mk(jnp.abs(rh_r)),
                   fm(0), fm(1), fm(2), fm(3), s2r]),
        jnp.zeros((3, 96), _f32)], axis=0)
    cps = (tmat, pmat)

    e0 = jnp.zeros((K, DEGW), _f32).at[:, 0].set(1.0)
    z16 = jnp.zeros((SCHUNK, DEGW), _f32)
    z64 = jnp.zeros((SCHUNK, HC), _f32)

    degp, eg3 = _sc_deg(dst3d, batch_i, e0, z16)
    eg2 = eg3.reshape(E, 1)

    h, ya, yb, rv = _tca0(x, degp[0], degp[1], W_emb, tile8(b_emb), w0h,
                          tile8(b0 + ts[0] * W0[0]))

    scs = []
    g = None
    for i in range(3):
        part0 = _sc_agg(jnp.stack([ya, yb]), src2, dst2, z64)
        y1a, y1b, filtp = _tcb(part0[0], part0[1], ya, yb, rv, Wf1,
                               tile8(bf1), Wf2, tile8(bf2), *cps, wo_perm,
                               W1, tile8(b1))
        fs2, fd2 = _sc_fsfd(filtp, src3, dst3)
        scs.append(_tcc(fs2, fd2, eg2, *cps))
        part1 = _sc_agg(jnp.stack([y1a, y1b]), src2, dst2, z64)
        dtv = (ts[i + 1] - ts[i]).reshape(1)
        if i < 2:
            h, ya, yb = _tca(h, part1[0], part1[1], y1a, y1b, rv, w0h,
                             tile8(b0 + ts[i + 1] * W0[0]), dtv)
        else:
            g = _tca3(h, part1[0], part1[1], y1a, y1b, rv,
                      batch_i.reshape(N, 1), Wr1, tile8(br1), Wr2,
                      tile8(br2), dtv)

    return _tcf(g, scs[0], scs[1], scs[2], wc1p, tile8(bc1), Wc2,
                tile8(bc2), Wc3, tile8(bc3))


# speedup vs baseline: 15.2461x; 1.7368x over previous
import functools

import jax
import jax.numpy as jnp
import numpy as np
from jax import lax
from jax.experimental import pallas as pl
from jax.experimental.pallas import tpu as pltpu
from jax.experimental.pallas import tpu_sc as plsc

N = 10000
E = 320000
D = 128
H = 128
HC = H // 2
F = 8
FH = 24
G = 8
NC = 10

NCORES = 2
NSUB = 16
NW = NCORES * NSUB
EW = E // NW
K = 80
NCH = EW // K
KA = 100
ES = E // NSUB
NCS = ES // KA
NBUF = 4
FNBUF = 5
STRIPE = N // NSUB
SCHUNK = 125
DEGW = 16
FW = 16
EROWS = E * FW // 128
RW = EW * FW // 128
KROWS = K * FW // 128

NB = 1000
BROWS = 2000

_f32 = jnp.float32
_i32 = jnp.int32

_MESH = plsc.VectorSubcoreMesh(
    core_axis_name="c", subcore_axis_name="s",
    num_cores=NCORES, num_subcores=NSUB)

_SC_PARAMS = pltpu.CompilerParams(
    use_tc_tiling_on_sc=False, needs_layout_passes=False)

_PERM = tuple((kp % 8) * 12 + (kp // 24) * 3 + ((kp % 24) // 8)
              for kp in range(96))

_PASS_FAMS = {0: (0,), 1: (0, 1), 2: (1,), 3: (2,), 4: (2, 3), 5: (3,)}

_LANE = np.arange(128)


def _const_p0p1():
    p0 = np.zeros((128, 128), np.float32)
    p1 = np.zeros((128, 128), np.float32)
    for lo in range(128):
        p0[(lo // 16) * 16 + (lo % 16) % 8, lo] = 1.0
        p1[(lo // 16) * 16 + 8 + (lo % 16) % 8, lo] = 1.0
    return p0, p1


_P0N, _P1N = _const_p0p1()
_DSLOTN = ((_LANE // 16)[:, None] == (_LANE // 16)[None, :]).astype(np.float32)
_C1N = np.array([[1.0 if l % 16 == g else 0.0 for l in range(128)]
                 for g in range(8)], np.float32)
_C2N = np.array([[1.0 if (l % 16 == qq) else 0.0 for qq in range(16)]
                 for l in range(128)], np.float32)


@functools.partial(
    pl.kernel,
    out_type=(jax.ShapeDtypeStruct((NCORES, N, DEGW), _f32),
              jax.ShapeDtypeStruct((EROWS, 128), _f32)),
    mesh=_MESH,
    compiler_params=_SC_PARAMS,
    scratch_types=[
        pltpu.VMEM((NCH, K), _i32),
        pltpu.VMEM((N,), _i32),
        pltpu.VMEM((K, DEGW), _f32),
        pltpu.VMEM((SCHUNK, DEGW), _f32),
        pltpu.VMEM((KROWS, 128), _f32),
        pltpu.VMEM_SHARED((N, DEGW), _f32),
        pltpu.SemaphoreType.DMA,
    ],
)
def _sc_deg(dst3_h, batch_h, e0_h, z16_h, degp_h, o_h,
            dstv, batchv, onesv, zb, ostage, degtab, dsem):
    ci = lax.axis_index("c")
    si = lax.axis_index("s")
    wid = ci * NSUB + si
    rbase = wid * RW
    pltpu.sync_copy(dst3_h.at[wid], dstv)
    pltpu.sync_copy(batch_h, batchv)
    pltpu.sync_copy(e0_h, onesv)
    pltpu.sync_copy(z16_h, zb)
    base = si * STRIPE
    for j in range(STRIPE // SCHUNK):
        pltpu.sync_copy(zb, degtab.at[pl.ds(base + j * SCHUNK, SCHUNK)])
    plsc.subcore_barrier()

    zeros16 = jnp.zeros((16,), _f32)
    ones16 = jnp.ones((16,), _f32)
    iota16 = lax.broadcasted_iota(_i32, (16,), 0)
    for rr in range(KROWS):
        for l in range(8):
            ostage[rr, pl.ds(l * 16, 16)] = zeros16

    def chunk(c, carry):
        pltpu.async_copy(onesv, degtab.at[dstv.at[c]], dsem, add=True)

        @pl.when(c >= NBUF)
        def _():
            pltpu.make_async_copy(onesv, degtab.at[dstv.at[c]], dsem).wait()

        idxs = []
        for j in range(K // 16):
            idx = dstv[c, pl.ds(j * 16, 16)]
            vals = plsc.load_gather(batchv, [idx])
            rowi = iota16 // 8 + 2 * j
            coli = (iota16 % 8) * 16 + vals
            plsc.store_scatter(ostage, [rowi, coli], ones16)
            idxs.append((rowi, coli))
        pltpu.sync_copy(ostage, o_h.at[pl.ds(rbase + c * KROWS, KROWS)])
        for rowi, coli in idxs:
            plsc.store_scatter(ostage, [rowi, coli], zeros16)
        return carry

    lax.fori_loop(0, NCH, chunk, 0)
    for _ in range(NBUF):
        pltpu.make_async_copy(onesv, degtab.at[dstv.at[0]], dsem).wait()
    plsc.subcore_barrier()
    for j in range(STRIPE // SCHUNK):
        st = base + j * SCHUNK
        pltpu.sync_copy(degtab.at[pl.ds(st, SCHUNK)],
                        degp_h.at[ci, pl.ds(st, SCHUNK)])


@functools.partial(
    pl.kernel,
    out_type=jax.ShapeDtypeStruct((NCORES, N, HC), _f32),
    mesh=_MESH,
    compiler_params=_SC_PARAMS,
    scratch_types=[
        pltpu.VMEM((NCS, KA), _i32),
        pltpu.VMEM((NCS, KA), _i32),
        [pltpu.VMEM((KA, HC), _f32)] * NBUF,
        pltpu.VMEM((SCHUNK, HC), _f32),
        pltpu.VMEM_SHARED((N, HC), _f32),
        [pltpu.SemaphoreType.DMA] * NBUF,
        [pltpu.SemaphoreType.DMA] * NBUF,
    ],
)
def _sc_agg(y2_h, src2_h, dst2_h, z64_h, part_h,
            srcv, dstv, rows, zb, acc, gsem, ssem):
    ci = lax.axis_index("c")
    si = lax.axis_index("s")
    yc_h = y2_h.at[ci]
    pltpu.sync_copy(src2_h.at[si], srcv)
    pltpu.sync_copy(dst2_h.at[si], dstv)
    pltpu.sync_copy(z64_h, zb)
    base = si * STRIPE
    for j in range(STRIPE // SCHUNK):
        pltpu.sync_copy(zb, acc.at[pl.ds(base + j * SCHUNK, SCHUNK)])
    plsc.subcore_barrier()

    for b in range(2):
        pltpu.async_copy(yc_h.at[srcv.at[b]], rows[b], gsem[b])

    def group(g, carry):
        for b in range(NBUF):
            c = g * NBUF + b
            b2 = (b + 2) % NBUF
            pltpu.make_async_copy(yc_h.at[srcv.at[c]], rows[b], gsem[b]).wait()

            @pl.when(c >= 2)
            def _():
                pltpu.make_async_copy(rows[b2], acc.at[dstv.at[c]],
                                      ssem[b2]).wait()

            @pl.when(c + 2 < NCS)
            def _():
                pltpu.async_copy(yc_h.at[srcv.at[c + 2]], rows[b2], gsem[b2])

            pltpu.async_copy(rows[b], acc.at[dstv.at[c]], ssem[b], add=True)
        return carry

    lax.fori_loop(0, NCS // NBUF, group, 0)
    for b in ((NCS - 2) % NBUF, (NCS - 1) % NBUF):
        pltpu.make_async_copy(rows[b], acc.at[dstv.at[0]], ssem[b]).wait()
    plsc.subcore_barrier()
    for j in range(STRIPE // SCHUNK):
        st = base + j * SCHUNK
        pltpu.sync_copy(acc.at[pl.ds(st, SCHUNK)],
                        part_h.at[ci, pl.ds(st, SCHUNK)])


@functools.partial(
    pl.kernel,
    out_type=jax.ShapeDtypeStruct((EROWS, 128), _f32),
    mesh=_MESH,
    compiler_params=_SC_PARAMS,
    scratch_types=[
        pltpu.VMEM((NCH, K), _i32),
        pltpu.VMEM((NCH, K), _i32),
        [pltpu.VMEM((K, FW), _f32)] * FNBUF,
        [pltpu.VMEM((K, FW), _f32)] * FNBUF,
        [pltpu.VMEM((KROWS, 128), _f32)] * FNBUF,
        [pltpu.SemaphoreType.DMA] * FNBUF,
        [pltpu.SemaphoreType.DMA] * FNBUF,
        [pltpu.SemaphoreType.DMA] * FNBUF,
    ],
)
def _sc_fsfd(filtp_h, src3_h, dst3_h, cmb_h,
             srcv, dstv, ra, rb, cst, gsa, gsb, ws):
    ci = lax.axis_index("c")
    si = lax.axis_index("s")
    wid = ci * NSUB + si
    rbase = wid * RW
    pltpu.sync_copy(src3_h.at[wid], srcv)
    pltpu.sync_copy(dst3_h.at[wid], dstv)
    mask8 = lax.broadcasted_iota(_i32, (16,), 0) < 8
    for b in range(2):
        pltpu.async_copy(filtp_h.at[srcv.at[b]], ra[b], gsa[b])
        pltpu.async_copy(filtp_h.at[dstv.at[b]], rb[b], gsb[b])

    def group(g, carry):
        for b in range(FNBUF):
            c = g * FNBUF + b
            b2 = (b + 2) % FNBUF
            pltpu.make_async_copy(filtp_h.at[srcv.at[c]], ra[b], gsa[b]).wait()
            pltpu.make_async_copy(filtp_h.at[dstv.at[c]], rb[b], gsb[b]).wait()

            @pl.when(c >= FNBUF)
            def _():
                pltpu.make_async_copy(cst[b], cmb_h.at[pl.ds(rbase, KROWS)],
                                      ws[b]).wait()

            for v in range(K):
                cv = jnp.where(mask8, ra[b][v, :], rb[b][v, :])
                cst[b][v // 8, pl.ds((v % 8) * 16, 16)] = cv

            @pl.when(c + 2 < NCH)
            def _():
                pltpu.async_copy(filtp_h.at[srcv.at[c + 2]], ra[b2], gsa[b2])
                pltpu.async_copy(filtp_h.at[dstv.at[c + 2]], rb[b2], gsb[b2])

            pltpu.async_copy(cst[b], cmb_h.at[pl.ds(rbase + c * KROWS, KROWS)],
                             ws[b])
        return carry

    lax.fori_loop(0, NCH // FNBUF, group, 0)
    for b in range(FNBUF):
        pltpu.make_async_copy(cst[b], cmb_h.at[pl.ds(rbase, KROWS)],
                              ws[b]).wait()


def _coord96(x096, x196, p_ref):
    A = p_ref[0:1, :]
    B0 = p_ref[1:2, :]
    B1 = p_ref[2:3, :]
    C0 = p_ref[3:4, :]
    C1 = p_ref[4:5, :]
    C2 = p_ref[5:6, :]
    Dc = p_ref[6:7, :]
    Dr = p_ref[7:8, :]
    mt = p_ref[8:9, :]
    mg = p_ref[9:10, :]
    ml = p_ref[10:11, :]
    mr = p_ref[11:12, :]
    s2r = p_ref[12:13, :]
    tri = jax.nn.relu(x196 - jnp.abs(x096 - A))
    gau = jnp.exp(((x096 - B0) ** 2 + (x196 - B1) ** 2) * s2r)
    lin = x096 * C0 + x196 * C1 + C2
    a = jnp.abs(x096 - Dc)
    rh = 1.0 / (1.0 + a) - 1.0 / (1.0 + jnp.abs(Dr - a))
    return mt * tri + mg * gau + ml * lin + mr * rh


def _dotT(a, b):
    return lax.dot_general(a, b, (((0,), (0,)), ((), ())),
                           preferred_element_type=_f32)


def _split_y(y, ya_ref, yb_ref):
    ya_ref[...] = y[:, 0:HC]
    yb_ref[...] = y[:, HC:H]


def _tca0_body(x_ref, da_ref, db_ref, we_ref, be_ref, w0_ref, b0_ref,
               h_ref, ya_ref, yb_ref, rv_ref):
    deg = da_ref[:, 0:1] + db_ref[:, 0:1] + 1.0
    r = 1.0 / jnp.sqrt(deg)
    h = jnp.dot(x_ref[...], we_ref[...], preferred_element_type=_f32) \
        + be_ref[0:1, :]
    y = (jnp.dot(h, w0_ref[...], preferred_element_type=_f32)
         + b0_ref[0:1, :]) * r
    h_ref[...] = h
    _split_y(y, ya_ref, yb_ref)
    rv_ref[...] = jnp.broadcast_to(r, (r.shape[0], 8))


def _gcn_out(pa_ref, pb_ref, ya_ref, yb_ref, r):
    return r * jnp.concatenate(
        [pa_ref[...] + ya_ref[...], pb_ref[...] + yb_ref[...]], axis=1)


def _tca_body(h_ref, pa_ref, pb_ref, ya_ref, yb_ref, rv_ref, w0_ref, b0_ref,
              dt_ref, ho_ref, yoa_ref, yob_ref):
    r = rv_ref[:, 0:1]
    h = h_ref[...] + dt_ref[0] * _gcn_out(pa_ref, pb_ref, ya_ref, yb_ref, r)
    y = (jnp.dot(h, w0_ref[...], preferred_element_type=_f32)
         + b0_ref[0:1, :]) * r
    ho_ref[...] = h
    _split_y(y, yoa_ref, yob_ref)


def _tca3_body(h_ref, pa_ref, pb_ref, ya_ref, yb_ref, rv_ref, bt_ref,
               wr1_ref, br1_ref, wr2_ref, br2_ref, dt_ref, g_ref):
    r = rv_ref[:, 0:1]
    h = h_ref[...] + dt_ref[0] * _gcn_out(pa_ref, pb_ref, ya_ref, yb_ref, r)
    t = jax.nn.relu(jnp.dot(h, wr1_ref[...], preferred_element_type=_f32)
                    + br1_ref[0:1, :])
    rr = jnp.dot(t, wr2_ref[...], preferred_element_type=_f32) + br2_ref[0:1, :]
    iot = lax.broadcasted_iota(_i32, (rr.shape[0], G), 1)
    oh = (bt_ref[...] == iot).astype(_f32)

    @pl.when(pl.program_id(0) == 0)
    def _():
        g_ref[...] = jnp.zeros(g_ref.shape, _f32)

    g_ref[...] += _dotT(oh, rr)


def _tcb_body(pa_ref, pb_ref, ya_ref, yb_ref, rv_ref, wf1_ref, bf1_ref,
              wf2_ref, bf2_ref, t_ref, p_ref, wo_ref, w1_ref, b1_ref,
              y1a_ref, y1b_ref, fp_ref):
    r = rv_ref[:, 0:1]
    z1 = _gcn_out(pa_ref, pb_ref, ya_ref, yb_ref, r)
    t1 = jax.nn.relu(jnp.dot(z1, wf1_ref[...], preferred_element_type=_f32)
                     + bf1_ref[0:1, :])
    filt = jnp.dot(t1, wf2_ref[...], preferred_element_type=_f32) \
        + bf2_ref[0:1, :]
    f96 = jnp.dot(filt, t_ref[...], preferred_element_type=_f32)
    c96 = _coord96(f96, f96, p_ref)
    h_out = z1 + jnp.dot(c96, wo_ref[...], preferred_element_type=_f32)
    y1 = (jnp.dot(h_out, w1_ref[...], preferred_element_type=_f32)
          + b1_ref[0:1, :]) * r
    _split_y(y1, y1a_ref, y1b_ref)
    fp_ref[...] = jnp.concatenate([filt, filt], axis=1)


def _tcc_body(cmb_ref, o_ref, p0_ref, p1_ref, d_ref, c1_ref, c2_ref, pp_ref,
              sc_ref):
    cmb = cmb_ref[...]
    ov = o_ref[...]
    fsr = jnp.dot(cmb, p0_ref[...], preferred_element_type=_f32)
    fdr = jnp.dot(cmb, p1_ref[...], preferred_element_type=_f32)
    x0 = jnp.minimum(fsr, fdr)
    x1 = jnp.maximum(fsr, fdr)

    @pl.when(pl.program_id(0) == 0)
    def _():
        sc_ref[...] = jnp.zeros(sc_ref.shape, _f32)

    for p in range(6):
        pr = lambda k: pp_ref[p * 16 + k:p * 16 + k + 1, :]
        feat = None
        for fid in _PASS_FAMS[p]:
            if fid == 0:
                v = jax.nn.relu(x1 - jnp.abs(x0 - pr(0)))
            elif fid == 1:
                v = jnp.exp(((x0 - pr(1)) ** 2 + (x1 - pr(2)) ** 2) * pr(12))
            elif fid == 2:
                v = x0 * pr(3) + x1 * pr(4) + pr(5)
            else:
                a = jnp.abs(x0 - pr(6))
                v = 1.0 / (1.0 + a) - 1.0 / (1.0 + jnp.abs(pr(7) - a))
            mv = pr(8 + fid) * v
            feat = mv if feat is None else feat + mv
        pm = _dotT(ov, feat) * d_ref[...]
        t1 = jnp.dot(c1_ref[...], pm, preferred_element_type=_f32)
        sp = jnp.dot(t1, c2_ref[...], preferred_element_type=_f32)
        sc_ref[:, p * 16:(p + 1) * 16] += sp

    colsum = _dotT(ov, jnp.ones((ov.shape[0], 1), _f32))
    cnt = jnp.dot(c1_ref[...], colsum, preferred_element_type=_f32)
    sc_ref[:, 96:97] += cnt


def _tcf_body(g_ref, s0_ref, s1_ref, s2_ref, wc1_ref, bc1_ref, wc2_ref,
              bc2_ref, wc3_ref, bc3_ref, out_ref):
    ph = jnp.zeros((G, 96), _f32)
    for s_ref in (s0_ref, s1_ref, s2_ref):
        ph = ph + s_ref[:, 0:96] / (s_ref[:, 96:97] + 1e-6)
    ph = ph * (1.0 / 3.0)
    z = jnp.concatenate([g_ref[...], ph], axis=1)
    z = jax.nn.relu(jnp.dot(z, wc1_ref[...], preferred_element_type=_f32)
                    + bc1_ref[0:1, :])
    z = jax.nn.relu(jnp.dot(z, wc2_ref[...], preferred_element_type=_f32)
                    + bc2_ref[0:1, :])
    out_ref[...] = jnp.dot(z, wc3_ref[...], preferred_element_type=_f32) \
        + bc3_ref[0:1, :]


def _node_spec(w):
    return pl.BlockSpec((NB, w), lambda i: (i, 0))


def _full_spec(shape):
    return pl.BlockSpec(shape, lambda i: tuple(0 for _ in shape))


def _full0(shape):
    return pl.BlockSpec(shape, lambda: tuple(0 for _ in shape))


_SMEM = pl.BlockSpec(memory_space=pltpu.SMEM)
_GRID_N = N // NB
_GRID_ER = EROWS // BROWS


def _nshape(w):
    return jax.ShapeDtypeStruct((N, w), _f32)


_tca0 = pl.pallas_call(
    _tca0_body,
    grid=(_GRID_N,),
    in_specs=[_node_spec(D), _node_spec(DEGW), _node_spec(DEGW),
              _full_spec((D, H)), _full_spec((8, H)),
              _full_spec((H, H)), _full_spec((8, H))],
    out_specs=[_node_spec(H), _node_spec(HC), _node_spec(HC), _node_spec(8)],
    out_shape=[_nshape(H), _nshape(HC), _nshape(HC), _nshape(8)],
)

_tca = pl.pallas_call(
    _tca_body,
    grid=(_GRID_N,),
    in_specs=[_node_spec(H), _node_spec(HC), _node_spec(HC), _node_spec(HC),
              _node_spec(HC), _node_spec(8),
              _full_spec((H, H)), _full_spec((8, H)), _SMEM],
    out_specs=[_node_spec(H), _node_spec(HC), _node_spec(HC)],
    out_shape=[_nshape(H), _nshape(HC), _nshape(HC)],
)

_tca3 = pl.pallas_call(
    _tca3_body,
    grid=(_GRID_N,),
    in_specs=[_node_spec(H), _node_spec(HC), _node_spec(HC), _node_spec(HC),
              _node_spec(HC), _node_spec(8),
              pl.BlockSpec((NB, 1), lambda i: (i, 0)),
              _full_spec((H, 2 * H)), _full_spec((8, 2 * H)),
              _full_spec((2 * H, H)), _full_spec((8, H)), _SMEM],
    out_specs=pl.BlockSpec((G, H), lambda i: (0, 0)),
    out_shape=jax.ShapeDtypeStruct((G, H), _f32),
)

_tcb = pl.pallas_call(
    _tcb_body,
    grid=(_GRID_N,),
    in_specs=[_node_spec(HC), _node_spec(HC), _node_spec(HC), _node_spec(HC),
              _node_spec(8),
              _full_spec((H, FH)), _full_spec((8, FH)),
              _full_spec((FH, F)), _full_spec((8, F)),
              _full_spec((F, 96)), _full_spec((16, 96)),
              _full_spec((96, H)), _full_spec((H, H)), _full_spec((8, H))],
    out_specs=[_node_spec(HC), _node_spec(HC), _node_spec(FW)],
    out_shape=[_nshape(HC), _nshape(HC), _nshape(FW)],
)

_tcc = pl.pallas_call(
    _tcc_body,
    grid=(_GRID_ER,),
    in_specs=[pl.BlockSpec((BROWS, 128), lambda i: (i, 0)),
              pl.BlockSpec((BROWS, 128), lambda i: (i, 0)),
              _full_spec((128, 128)), _full_spec((128, 128)),
              _full_spec((128, 128)), _full_spec((8, 128)),
              _full_spec((128, 16)), _full_spec((96, 128))],
    out_specs=pl.BlockSpec((G, 128), lambda i: (0, 0)),
    out_shape=jax.ShapeDtypeStruct((G, 128), _f32),
)

_tcf = pl.pallas_call(
    _tcf_body,
    in_specs=[_full0((G, H)), _full0((G, 128)), _full0((G, 128)),
              _full0((G, 128)),
              _full0((H + 96, H // 2)), _full0((8, H // 2)),
              _full0((H // 2, H // 4)), _full0((8, H // 4)),
              _full0((H // 4, NC)), _full0((8, NC))],
    out_specs=_full0((G, NC)),
    out_shape=jax.ShapeDtypeStruct((G, NC), _f32),
)


def kernel(x, edge_index, batch, W_emb, b_emb, W0, b0, W1, b1, Wf1, bf1,
           Wf2, bf2, tri_t, g_mu, g_sig, l_m, l_b, rh_c, rh_r, W_out0,
           Wr1, br1, Wr2, br2, Wc1, bc1, Wc2, bc2, Wc3, bc3):
    src = edge_index[0].astype(_i32)
    dst = edge_index[1].astype(_i32)
    src2 = src.reshape(NSUB, NCS, KA)
    dst2 = dst.reshape(NSUB, NCS, KA)
    src3d = src.reshape(NW, NCH, K)
    dst3d = dst.reshape(NW, NCH, K)
    batch_i = batch.astype(_i32)

    ts = jnp.linspace(0.0, 1.0, 4)
    w0h = W0[1:]
    perm = jnp.asarray(_PERM, dtype=_i32)
    wo_perm = W_out0[perm]
    wc1p = jnp.concatenate([Wc1[:H], Wc1[H:][perm]], axis=0)
    tile8 = lambda b: jnp.tile(b.reshape(1, -1), (8, 1))

    kp = np.arange(96)
    jvn = (kp % 24) // 8
    famn = kp // 24
    jv = jnp.asarray(jvn)
    tabs = [tri_t[jv], g_mu[jv, 0], g_mu[jv, 1], l_m[jv, 0], l_m[jv, 1],
            l_b[jv], rh_c[jv], jnp.abs(rh_r)[jv],
            jnp.asarray((famn == 0).astype(np.float32)),
            jnp.asarray((famn == 1).astype(np.float32)),
            jnp.asarray((famn == 2).astype(np.float32)),
            jnp.asarray((famn == 3).astype(np.float32)),
            jnp.full((96,), -1.0, _f32) / (2.0 * g_sig * g_sig)]

    pmat = jnp.concatenate([jnp.stack(tabs), jnp.zeros((3, 96), _f32)], axis=0)
    tmatn = (kp[None, :] % F == np.arange(F)[:, None]).astype(np.float32)
    tmat = jnp.asarray(tmatn)

    p0c, p1c = jnp.asarray(_P0N), jnp.asarray(_P1N)
    dsl, c1c, c2c = jnp.asarray(_DSLOTN), jnp.asarray(_C1N), jnp.asarray(_C2N)
    idx6 = jnp.asarray(np.arange(6)[:, None] * 16 + (_LANE[None, :] % 16))
    pp = jnp.concatenate(
        [jnp.stack([t[idx6] for t in tabs], axis=1),
         jnp.zeros((6, 3, 128), _f32)], axis=1).reshape(96, 128)

    e0 = jnp.zeros((K, DEGW), _f32).at[:, 0].set(1.0)
    z16 = jnp.zeros((SCHUNK, DEGW), _f32)
    z64 = jnp.zeros((SCHUNK, HC), _f32)

    degp, obig = _sc_deg(dst3d, batch_i, e0, z16)

    h, ya, yb, rv = _tca0(x, degp[0], degp[1], W_emb, tile8(b_emb), w0h,
                          tile8(b0 + ts[0] * W0[0]))

    scs = []
    g = None
    for i in range(3):
        part0 = _sc_agg(jnp.stack([ya, yb]), src2, dst2, z64)
        y1a, y1b, filtp = _tcb(part0[0], part0[1], ya, yb, rv, Wf1,
                               tile8(bf1), Wf2, tile8(bf2), tmat, pmat,
                               wo_perm, W1, tile8(b1))
        cmb = _sc_fsfd(filtp, src3d, dst3d)
        scs.append(_tcc(cmb, obig, p0c, p1c, dsl, c1c, c2c, pp))
        part1 = _sc_agg(jnp.stack([y1a, y1b]), src2, dst2, z64)
        dtv = (ts[i + 1] - ts[i]).reshape(1)
        if i < 2:
            h, ya, yb = _tca(h, part1[0], part1[1], y1a, y1b, rv, w0h,
                             tile8(b0 + ts[i + 1] * W0[0]), dtv)
        else:
            g = _tca3(h, part1[0], part1[1], y1a, y1b, rv,
                      batch_i.reshape(N, 1), Wr1, tile8(br1), Wr2,
                      tile8(br2), dtv)

    return _tcf(g, scs[0], scs[1], scs[2], wc1p, tile8(bc1), Wc2,
                tile8(bc2), Wc3, tile8(bc3))


# speedup vs baseline: 15.7919x; 1.0358x over previous
import functools

import jax
import jax.numpy as jnp
import numpy as np
from jax import lax
from jax.experimental import pallas as pl
from jax.experimental.pallas import tpu as pltpu
from jax.experimental.pallas import tpu_sc as plsc

N = 10000
E = 320000
D = 128
H = 128
HC = H // 2
F = 8
FH = 24
G = 8
NC = 10

NCORES = 2
NSUB = 16
NW = NCORES * NSUB
EW = E // NW
K = 80
NCH = EW // K
KA = 125
ES = E // NSUB
NCS = ES // KA
NBUF = 4
FNBUF = 5
STRIPE = N // NSUB
SCHUNK = 125
DEGW = 16
FW = 16
EROWS = E * FW // 128
RW = EW * FW // 128
KROWS = K * FW // 128

NB = 1000
BROWS = 2000

_f32 = jnp.float32
_i32 = jnp.int32

_MESH = plsc.VectorSubcoreMesh(
    core_axis_name="c", subcore_axis_name="s",
    num_cores=NCORES, num_subcores=NSUB)

_SC_PARAMS = pltpu.CompilerParams(
    use_tc_tiling_on_sc=False, needs_layout_passes=False)

_PERM = tuple((kp % 8) * 12 + (kp // 24) * 3 + ((kp % 24) // 8)
              for kp in range(96))

_PASS_FAMS = {0: (0,), 1: (0, 1), 2: (1,), 3: (2,), 4: (2, 3), 5: (3,)}

_LANE = np.arange(128)


def _const_p0p1():
    p0 = np.zeros((128, 128), np.float32)
    p1 = np.zeros((128, 128), np.float32)
    for lo in range(128):
        p0[(lo // 16) * 16 + (lo % 16) % 8, lo] = 1.0
        p1[(lo // 16) * 16 + 8 + (lo % 16) % 8, lo] = 1.0
    return p0, p1


_P0N, _P1N = _const_p0p1()
_DSLOTN = ((_LANE // 16)[:, None] == (_LANE // 16)[None, :]).astype(np.float32)
_C1N = np.array([[1.0 if l % 16 == g else 0.0 for l in range(128)]
                 for g in range(8)], np.float32)
_C2N = np.array([[1.0 if (l % 16 == qq) else 0.0 for qq in range(16)]
                 for l in range(128)], np.float32)


@functools.partial(
    pl.kernel,
    out_type=(jax.ShapeDtypeStruct((NCORES, N, DEGW), _f32),
              jax.ShapeDtypeStruct((EROWS, 128), _f32)),
    mesh=_MESH,
    compiler_params=_SC_PARAMS,
    scratch_types=[
        pltpu.VMEM((NCH, K), _i32),
        pltpu.VMEM((N,), _i32),
        pltpu.VMEM((K, DEGW), _f32),
        pltpu.VMEM((SCHUNK, DEGW), _f32),
        [pltpu.VMEM((KROWS, 128), _f32)] * 2,
        pltpu.VMEM_SHARED((N, DEGW), _f32),
        pltpu.SemaphoreType.DMA,
        [pltpu.SemaphoreType.DMA] * 2,
    ],
)
def _sc_deg(dst3_h, batch_h, e0_h, z16_h, degp_h, o_h,
            dstv, batchv, onesv, zb, ostage, degtab, dsem, osem):
    ci = lax.axis_index("c")
    si = lax.axis_index("s")
    wid = ci * NSUB + si
    rbase = wid * RW
    pltpu.sync_copy(dst3_h.at[wid], dstv)
    pltpu.sync_copy(batch_h, batchv)
    pltpu.sync_copy(e0_h, onesv)
    pltpu.sync_copy(z16_h, zb)
    base = si * STRIPE
    for j in range(STRIPE // SCHUNK):
        pltpu.sync_copy(zb, degtab.at[pl.ds(base + j * SCHUNK, SCHUNK)])
    plsc.subcore_barrier()

    zeros16 = jnp.zeros((16,), _f32)
    ones16 = jnp.ones((16,), _f32)
    iota16 = lax.broadcasted_iota(_i32, (16,), 0)
    for b in range(2):
        for rr in range(KROWS):
            for l in range(8):
                ostage[b][rr, pl.ds(l * 16, 16)] = zeros16

    def group2(g2, carry):
        for b in range(2):
            c = g2 * 2 + b
            pltpu.async_copy(onesv, degtab.at[dstv.at[c]], dsem, add=True)

            @pl.when(c >= NBUF)
            def _():
                pltpu.make_async_copy(onesv, degtab.at[dstv.at[c]],
                                      dsem).wait()

            @pl.when(c >= 2)
            def _():
                pltpu.make_async_copy(ostage[b],
                                      o_h.at[pl.ds(rbase, KROWS)],
                                      osem[b]).wait()
                for rr in range(KROWS):
                    for l in range(8):
                        ostage[b][rr, pl.ds(l * 16, 16)] = zeros16

            for j in range(K // 16):
                idx = dstv[c, pl.ds(j * 16, 16)]
                vals = plsc.load_gather(batchv, [idx])
                rowi = iota16 // 8 + 2 * j
                coli = (iota16 % 8) * 16 + vals
                plsc.store_scatter(ostage[b], [rowi, coli], ones16)
            pltpu.async_copy(ostage[b], o_h.at[pl.ds(rbase + c * KROWS, KROWS)],
                             osem[b])
        return carry

    lax.fori_loop(0, NCH // 2, group2, 0)
    ct = NCH - 1
    pltpu.async_copy(onesv, degtab.at[dstv.at[ct]], dsem, add=True)
    pltpu.make_async_copy(ostage[0], o_h.at[pl.ds(rbase, KROWS)],
                          osem[0]).wait()
    for rr in range(KROWS):
        for l in range(8):
            ostage[0][rr, pl.ds(l * 16, 16)] = zeros16
    for j in range(K // 16):
        idx = dstv[ct, pl.ds(j * 16, 16)]
        vals = plsc.load_gather(batchv, [idx])
        rowi = iota16 // 8 + 2 * j
        coli = (iota16 % 8) * 16 + vals
        plsc.store_scatter(ostage[0], [rowi, coli], ones16)
    pltpu.async_copy(ostage[0], o_h.at[pl.ds(rbase + ct * KROWS, KROWS)],
                     osem[0])
    pltpu.make_async_copy(ostage[0], o_h.at[pl.ds(rbase, KROWS)],
                          osem[0]).wait()
    pltpu.make_async_copy(ostage[1], o_h.at[pl.ds(rbase, KROWS)],
                          osem[1]).wait()
    for _ in range(NBUF + 1):
        pltpu.make_async_copy(onesv, degtab.at[dstv.at[0]], dsem).wait()
    plsc.subcore_barrier()
    for j in range(STRIPE // SCHUNK):
        st = base + j * SCHUNK
        pltpu.sync_copy(degtab.at[pl.ds(st, SCHUNK)],
                        degp_h.at[ci, pl.ds(st, SCHUNK)])


@functools.partial(
    pl.kernel,
    out_type=jax.ShapeDtypeStruct((NCORES, N, HC), _f32),
    mesh=_MESH,
    compiler_params=_SC_PARAMS,
    scratch_types=[
        pltpu.VMEM((NCS, KA), _i32),
        pltpu.VMEM((NCS, KA), _i32),
        [pltpu.VMEM((KA, HC), _f32)] * NBUF,
        pltpu.VMEM((SCHUNK, HC), _f32),
        pltpu.VMEM_SHARED((N, HC), _f32),
        [pltpu.SemaphoreType.DMA] * NBUF,
        [pltpu.SemaphoreType.DMA] * NBUF,
    ],
)
def _sc_agg(y2_h, src2_h, dst2_h, z64_h, part_h,
            srcv, dstv, rows, zb, acc, gsem, ssem):
    ci = lax.axis_index("c")
    si = lax.axis_index("s")
    yc_h = y2_h.at[ci]
    pltpu.sync_copy(src2_h.at[si], srcv)
    pltpu.sync_copy(dst2_h.at[si], dstv)
    pltpu.sync_copy(z64_h, zb)
    base = si * STRIPE
    for j in range(STRIPE // SCHUNK):
        pltpu.sync_copy(zb, acc.at[pl.ds(base + j * SCHUNK, SCHUNK)])
    plsc.subcore_barrier()

    for b in range(2):
        pltpu.async_copy(yc_h.at[srcv.at[b]], rows[b], gsem[b])

    def group(g, carry):
        for b in range(NBUF):
            c = g * NBUF + b
            b2 = (b + 2) % NBUF
            pltpu.make_async_copy(yc_h.at[srcv.at[c]], rows[b], gsem[b]).wait()

            @pl.when(c >= 2)
            def _():
                pltpu.make_async_copy(rows[b2], acc.at[dstv.at[c]],
                                      ssem[b2]).wait()

            @pl.when(c + 2 < NCS)
            def _():
                pltpu.async_copy(yc_h.at[srcv.at[c + 2]], rows[b2], gsem[b2])

            pltpu.async_copy(rows[b], acc.at[dstv.at[c]], ssem[b], add=True)
        return carry

    lax.fori_loop(0, NCS // NBUF, group, 0)
    for b in ((NCS - 2) % NBUF, (NCS - 1) % NBUF):
        pltpu.make_async_copy(rows[b], acc.at[dstv.at[0]], ssem[b]).wait()
    plsc.subcore_barrier()
    for j in range(STRIPE // SCHUNK):
        st = base + j * SCHUNK
        pltpu.sync_copy(acc.at[pl.ds(st, SCHUNK)],
                        part_h.at[ci, pl.ds(st, SCHUNK)])


@functools.partial(
    pl.kernel,
    out_type=jax.ShapeDtypeStruct((EROWS, 128), _f32),
    mesh=_MESH,
    compiler_params=_SC_PARAMS,
    scratch_types=[
        pltpu.VMEM((NCH, K), _i32),
        pltpu.VMEM((NCH, K), _i32),
        [pltpu.VMEM((K, FW), _f32)] * FNBUF,
        [pltpu.VMEM((K, FW), _f32)] * FNBUF,
        [pltpu.VMEM((KROWS, 128), _f32)] * FNBUF,
        [pltpu.SemaphoreType.DMA] * FNBUF,
        [pltpu.SemaphoreType.DMA] * FNBUF,
        [pltpu.SemaphoreType.DMA] * FNBUF,
    ],
)
def _sc_fsfd(filtp_h, src3_h, dst3_h, cmb_h,
             srcv, dstv, ra, rb, cst, gsa, gsb, ws):
    ci = lax.axis_index("c")
    si = lax.axis_index("s")
    wid = ci * NSUB + si
    rbase = wid * RW
    pltpu.sync_copy(src3_h.at[wid], srcv)
    pltpu.sync_copy(dst3_h.at[wid], dstv)
    mask8 = lax.broadcasted_iota(_i32, (16,), 0) < 8
    for b in range(2):
        pltpu.async_copy(filtp_h.at[srcv.at[b]], ra[b], gsa[b])
        pltpu.async_copy(filtp_h.at[dstv.at[b]], rb[b], gsb[b])

    def group(g, carry):
        for b in range(FNBUF):
            c = g * FNBUF + b
            b2 = (b + 2) % FNBUF
            pltpu.make_async_copy(filtp_h.at[srcv.at[c]], ra[b], gsa[b]).wait()
            pltpu.make_async_copy(filtp_h.at[dstv.at[c]], rb[b], gsb[b]).wait()

            @pl.when(c >= FNBUF)
            def _():
                pltpu.make_async_copy(cst[b], cmb_h.at[pl.ds(rbase, KROWS)],
                                      ws[b]).wait()

            for v in range(K):
                cv = jnp.where(mask8, ra[b][v, :], rb[b][v, :])
                cst[b][v // 8, pl.ds((v % 8) * 16, 16)] = cv

            @pl.when(c + 2 < NCH)
            def _():
                pltpu.async_copy(filtp_h.at[srcv.at[c + 2]], ra[b2], gsa[b2])
                pltpu.async_copy(filtp_h.at[dstv.at[c + 2]], rb[b2], gsb[b2])

            pltpu.async_copy(cst[b], cmb_h.at[pl.ds(rbase + c * KROWS, KROWS)],
                             ws[b])
        return carry

    lax.fori_loop(0, NCH // FNBUF, group, 0)
    for b in range(FNBUF):
        pltpu.make_async_copy(cst[b], cmb_h.at[pl.ds(rbase, KROWS)],
                              ws[b]).wait()


def _coord96(x096, x196, p_ref):
    A = p_ref[0:1, :]
    B0 = p_ref[1:2, :]
    B1 = p_ref[2:3, :]
    C0 = p_ref[3:4, :]
    C1 = p_ref[4:5, :]
    C2 = p_ref[5:6, :]
    Dc = p_ref[6:7, :]
    Dr = p_ref[7:8, :]
    mt = p_ref[8:9, :]
    mg = p_ref[9:10, :]
    ml = p_ref[10:11, :]
    mr = p_ref[11:12, :]
    s2r = p_ref[12:13, :]
    tri = jax.nn.relu(x196 - jnp.abs(x096 - A))
    gau = jnp.exp(((x096 - B0) ** 2 + (x196 - B1) ** 2) * s2r)
    lin = x096 * C0 + x196 * C1 + C2
    a = jnp.abs(x096 - Dc)
    rh = 1.0 / (1.0 + a) - 1.0 / (1.0 + jnp.abs(Dr - a))
    return mt * tri + mg * gau + ml * lin + mr * rh


def _dotT(a, b):
    return lax.dot_general(a, b, (((0,), (0,)), ((), ())),
                           preferred_element_type=_f32)


def _split_y(y, ya_ref, yb_ref):
    ya_ref[...] = y[:, 0:HC]
    yb_ref[...] = y[:, HC:H]


def _tca0_body(x_ref, da_ref, db_ref, we_ref, be_ref, w0_ref, b0_ref,
               h_ref, ya_ref, yb_ref, rv_ref):
    deg = da_ref[:, 0:1] + db_ref[:, 0:1] + 1.0
    r = 1.0 / jnp.sqrt(deg)
    h = jnp.dot(x_ref[...], we_ref[...], preferred_element_type=_f32) \
        + be_ref[0:1, :]
    y = (jnp.dot(h, w0_ref[...], preferred_element_type=_f32)
         + b0_ref[0:1, :]) * r
    h_ref[...] = h
    _split_y(y, ya_ref, yb_ref)
    rv_ref[...] = jnp.broadcast_to(r, (r.shape[0], 8))


def _gcn_out(pa_ref, pb_ref, ya_ref, yb_ref, r):
    return r * jnp.concatenate(
        [pa_ref[...] + ya_ref[...], pb_ref[...] + yb_ref[...]], axis=1)


def _tca_body(h_ref, pa_ref, pb_ref, ya_ref, yb_ref, rv_ref, w0_ref, b0_ref,
              dt_ref, ho_ref, yoa_ref, yob_ref):
    r = rv_ref[:, 0:1]
    h = h_ref[...] + dt_ref[0] * _gcn_out(pa_ref, pb_ref, ya_ref, yb_ref, r)
    y = (jnp.dot(h, w0_ref[...], preferred_element_type=_f32)
         + b0_ref[0:1, :]) * r
    ho_ref[...] = h
    _split_y(y, yoa_ref, yob_ref)


def _tca3_body(h_ref, pa_ref, pb_ref, ya_ref, yb_ref, rv_ref, bt_ref,
               wr1_ref, br1_ref, wr2_ref, br2_ref, dt_ref, g_ref):
    r = rv_ref[:, 0:1]
    h = h_ref[...] + dt_ref[0] * _gcn_out(pa_ref, pb_ref, ya_ref, yb_ref, r)
    t = jax.nn.relu(jnp.dot(h, wr1_ref[...], preferred_element_type=_f32)
                    + br1_ref[0:1, :])
    rr = jnp.dot(t, wr2_ref[...], preferred_element_type=_f32) + br2_ref[0:1, :]
    iot = lax.broadcasted_iota(_i32, (rr.shape[0], G), 1)
    oh = (bt_ref[...] == iot).astype(_f32)

    @pl.when(pl.program_id(0) == 0)
    def _():
        g_ref[...] = jnp.zeros(g_ref.shape, _f32)

    g_ref[...] += _dotT(oh, rr)


def _tcb_body(pa_ref, pb_ref, ya_ref, yb_ref, rv_ref, wf1_ref, bf1_ref,
              wf2_ref, bf2_ref, t_ref, p_ref, wo_ref, w1_ref, b1_ref,
              y1a_ref, y1b_ref, fp_ref):
    r = rv_ref[:, 0:1]
    z1 = _gcn_out(pa_ref, pb_ref, ya_ref, yb_ref, r)
    t1 = jax.nn.relu(jnp.dot(z1, wf1_ref[...], preferred_element_type=_f32)
                     + bf1_ref[0:1, :])
    filt = jnp.dot(t1, wf2_ref[...], preferred_element_type=_f32) \
        + bf2_ref[0:1, :]
    f96 = jnp.dot(filt, t_ref[...], preferred_element_type=_f32)
    c96 = _coord96(f96, f96, p_ref)
    h_out = z1 + jnp.dot(c96, wo_ref[...], preferred_element_type=_f32)
    y1 = (jnp.dot(h_out, w1_ref[...], preferred_element_type=_f32)
          + b1_ref[0:1, :]) * r
    _split_y(y1, y1a_ref, y1b_ref)
    fp_ref[...] = jnp.concatenate([filt, filt], axis=1)


def _tcc_body(cmb_ref, o_ref, p0_ref, p1_ref, d_ref, c1_ref, c2_ref, pp_ref,
              sc_ref):
    cmb = cmb_ref[...]
    ov = o_ref[...]
    fsr = jnp.dot(cmb, p0_ref[...], preferred_element_type=_f32)
    fdr = jnp.dot(cmb, p1_ref[...], preferred_element_type=_f32)
    x0 = jnp.minimum(fsr, fdr)
    x1 = jnp.maximum(fsr, fdr)

    @pl.when(pl.program_id(0) == 0)
    def _():
        sc_ref[...] = jnp.zeros(sc_ref.shape, _f32)

    for p in range(6):
        pr = lambda k: pp_ref[p * 16 + k:p * 16 + k + 1, :]
        feat = None
        for fid in _PASS_FAMS[p]:
            if fid == 0:
                v = jax.nn.relu(x1 - jnp.abs(x0 - pr(0)))
            elif fid == 1:
                v = jnp.exp(((x0 - pr(1)) ** 2 + (x1 - pr(2)) ** 2) * pr(12))
            elif fid == 2:
                v = x0 * pr(3) + x1 * pr(4) + pr(5)
            else:
                a = jnp.abs(x0 - pr(6))
                v = 1.0 / (1.0 + a) - 1.0 / (1.0 + jnp.abs(pr(7) - a))
            mv = pr(8 + fid) * v
            feat = mv if feat is None else feat + mv
        pm = _dotT(ov, feat) * d_ref[...]
        t1 = jnp.dot(c1_ref[...], pm, preferred_element_type=_f32)
        sp = jnp.dot(t1, c2_ref[...], preferred_element_type=_f32)
        sc_ref[:, p * 16:(p + 1) * 16] += sp

    colsum = _dotT(ov, jnp.ones((ov.shape[0], 1), _f32))
    cnt = jnp.dot(c1_ref[...], colsum, preferred_element_type=_f32)
    sc_ref[:, 96:97] += cnt


def _tcf_body(g_ref, s0_ref, s1_ref, s2_ref, wc1_ref, bc1_ref, wc2_ref,
              bc2_ref, wc3_ref, bc3_ref, out_ref):
    ph = jnp.zeros((G, 96), _f32)
    for s_ref in (s0_ref, s1_ref, s2_ref):
        ph = ph + s_ref[:, 0:96] / (s_ref[:, 96:97] + 1e-6)
    ph = ph * (1.0 / 3.0)
    z = jnp.concatenate([g_ref[...], ph], axis=1)
    z = jax.nn.relu(jnp.dot(z, wc1_ref[...], preferred_element_type=_f32)
                    + bc1_ref[0:1, :])
    z = jax.nn.relu(jnp.dot(z, wc2_ref[...], preferred_element_type=_f32)
                    + bc2_ref[0:1, :])
    out_ref[...] = jnp.dot(z, wc3_ref[...], preferred_element_type=_f32) \
        + bc3_ref[0:1, :]


def _node_spec(w):
    return pl.BlockSpec((NB, w), lambda i: (i, 0))


def _full_spec(shape):
    return pl.BlockSpec(shape, lambda i: tuple(0 for _ in shape))


def _full0(shape):
    return pl.BlockSpec(shape, lambda: tuple(0 for _ in shape))


_SMEM = pl.BlockSpec(memory_space=pltpu.SMEM)
_GRID_N = N // NB
_GRID_ER = EROWS // BROWS


def _nshape(w):
    return jax.ShapeDtypeStruct((N, w), _f32)


_tca0 = pl.pallas_call(
    _tca0_body,
    grid=(_GRID_N,),
    in_specs=[_node_spec(D), _node_spec(DEGW), _node_spec(DEGW),
              _full_spec((D, H)), _full_spec((8, H)),
              _full_spec((H, H)), _full_spec((8, H))],
    out_specs=[_node_spec(H), _node_spec(HC), _node_spec(HC), _node_spec(8)],
    out_shape=[_nshape(H), _nshape(HC), _nshape(HC), _nshape(8)],
)

_tca = pl.pallas_call(
    _tca_body,
    grid=(_GRID_N,),
    in_specs=[_node_spec(H), _node_spec(HC), _node_spec(HC), _node_spec(HC),
              _node_spec(HC), _node_spec(8),
              _full_spec((H, H)), _full_spec((8, H)), _SMEM],
    out_specs=[_node_spec(H), _node_spec(HC), _node_spec(HC)],
    out_shape=[_nshape(H), _nshape(HC), _nshape(HC)],
)

_tca3 = pl.pallas_call(
    _tca3_body,
    grid=(_GRID_N,),
    in_specs=[_node_spec(H), _node_spec(HC), _node_spec(HC), _node_spec(HC),
              _node_spec(HC), _node_spec(8),
              pl.BlockSpec((NB, 1), lambda i: (i, 0)),
              _full_spec((H, 2 * H)), _full_spec((8, 2 * H)),
              _full_spec((2 * H, H)), _full_spec((8, H)), _SMEM],
    out_specs=pl.BlockSpec((G, H), lambda i: (0, 0)),
    out_shape=jax.ShapeDtypeStruct((G, H), _f32),
)

_tcb = pl.pallas_call(
    _tcb_body,
    grid=(_GRID_N,),
    in_specs=[_node_spec(HC), _node_spec(HC), _node_spec(HC), _node_spec(HC),
              _node_spec(8),
              _full_spec((H, FH)), _full_spec((8, FH)),
              _full_spec((FH, F)), _full_spec((8, F)),
              _full_spec((F, 96)), _full_spec((16, 96)),
              _full_spec((96, H)), _full_spec((H, H)), _full_spec((8, H))],
    out_specs=[_node_spec(HC), _node_spec(HC), _node_spec(FW)],
    out_shape=[_nshape(HC), _nshape(HC), _nshape(FW)],
)

_tcc = pl.pallas_call(
    _tcc_body,
    grid=(_GRID_ER,),
    in_specs=[pl.BlockSpec((BROWS, 128), lambda i: (i, 0)),
              pl.BlockSpec((BROWS, 128), lambda i: (i, 0)),
              _full_spec((128, 128)), _full_spec((128, 128)),
              _full_spec((128, 128)), _full_spec((8, 128)),
              _full_spec((128, 16)), _full_spec((96, 128))],
    out_specs=pl.BlockSpec((G, 128), lambda i: (0, 0)),
    out_shape=jax.ShapeDtypeStruct((G, 128), _f32),
)

_tcf = pl.pallas_call(
    _tcf_body,
    in_specs=[_full0((G, H)), _full0((G, 128)), _full0((G, 128)),
              _full0((G, 128)),
              _full0((H + 96, H // 2)), _full0((8, H // 2)),
              _full0((H // 2, H // 4)), _full0((8, H // 4)),
              _full0((H // 4, NC)), _full0((8, NC))],
    out_specs=_full0((G, NC)),
    out_shape=jax.ShapeDtypeStruct((G, NC), _f32),
)


def kernel(x, edge_index, batch, W_emb, b_emb, W0, b0, W1, b1, Wf1, bf1,
           Wf2, bf2, tri_t, g_mu, g_sig, l_m, l_b, rh_c, rh_r, W_out0,
           Wr1, br1, Wr2, br2, Wc1, bc1, Wc2, bc2, Wc3, bc3):
    src = edge_index[0].astype(_i32)
    dst = edge_index[1].astype(_i32)
    src2 = src.reshape(NSUB, NCS, KA)
    dst2 = dst.reshape(NSUB, NCS, KA)
    src3d = src.reshape(NW, NCH, K)
    dst3d = dst.reshape(NW, NCH, K)
    batch_i = batch.astype(_i32)

    ts = jnp.linspace(0.0, 1.0, 4)
    w0h = W0[1:]
    perm = jnp.asarray(_PERM, dtype=_i32)
    wo_perm = W_out0[perm]
    wc1p = jnp.concatenate([Wc1[:H], Wc1[H:][perm]], axis=0)
    tile8 = lambda b: jnp.tile(b.reshape(1, -1), (8, 1))

    kp = np.arange(96)
    jvn = (kp % 24) // 8
    famn = kp // 24
    jv = jnp.asarray(jvn)
    tabs = [tri_t[jv], g_mu[jv, 0], g_mu[jv, 1], l_m[jv, 0], l_m[jv, 1],
            l_b[jv], rh_c[jv], jnp.abs(rh_r)[jv],
            jnp.asarray((famn == 0).astype(np.float32)),
            jnp.asarray((famn == 1).astype(np.float32)),
            jnp.asarray((famn == 2).astype(np.float32)),
            jnp.asarray((famn == 3).astype(np.float32)),
            jnp.full((96,), -1.0, _f32) / (2.0 * g_sig * g_sig)]

    pmat = jnp.concatenate([jnp.stack(tabs), jnp.zeros((3, 96), _f32)], axis=0)
    tmatn = (kp[None, :] % F == np.arange(F)[:, None]).astype(np.float32)
    tmat = jnp.asarray(tmatn)

    p0c, p1c = jnp.asarray(_P0N), jnp.asarray(_P1N)
    dsl, c1c, c2c = jnp.asarray(_DSLOTN), jnp.asarray(_C1N), jnp.asarray(_C2N)
    idx6 = jnp.asarray(np.arange(6)[:, None] * 16 + (_LANE[None, :] % 16))
    pp = jnp.concatenate(
        [jnp.stack([t[idx6] for t in tabs], axis=1),
         jnp.zeros((6, 3, 128), _f32)], axis=1).reshape(96, 128)

    e0 = jnp.zeros((K, DEGW), _f32).at[:, 0].set(1.0)
    z16 = jnp.zeros((SCHUNK, DEGW), _f32)
    z64 = jnp.zeros((SCHUNK, HC), _f32)

    degp, obig = _sc_deg(dst3d, batch_i, e0, z16)

    h, ya, yb, rv = _tca0(x, degp[0], degp[1], W_emb, tile8(b_emb), w0h,
                          tile8(b0 + ts[0] * W0[0]))

    scs = []
    g = None
    for i in range(3):
        part0 = _sc_agg(jnp.stack([ya, yb]), src2, dst2, z64)
        y1a, y1b, filtp = _tcb(part0[0], part0[1], ya, yb, rv, Wf1,
                               tile8(bf1), Wf2, tile8(bf2), tmat, pmat,
                               wo_perm, W1, tile8(b1))
        cmb = _sc_fsfd(filtp, src3d, dst3d)
        scs.append(_tcc(cmb, obig, p0c, p1c, dsl, c1c, c2c, pp))
        part1 = _sc_agg(jnp.stack([y1a, y1b]), src2, dst2, z64)
        dtv = (ts[i + 1] - ts[i]).reshape(1)
        if i < 2:
            h, ya, yb = _tca(h, part1[0], part1[1], y1a, y1b, rv, w0h,
                             tile8(b0 + ts[i + 1] * W0[0]), dtv)
        else:
            g = _tca3(h, part1[0], part1[1], y1a, y1b, rv,
                      batch_i.reshape(N, 1), Wr1, tile8(br1), Wr2,
                      tile8(br2), dtv)

    return _tcf(g, scs[0], scs[1], scs[2], wc1p, tile8(bc1), Wc2,
                tile8(bc2), Wc3, tile8(bc3))
